# Initial kernel scaffold; baseline (speedup 1.0000x reference)
import jax
import jax.numpy as jnp
from jax.experimental import pallas as pl


def kernel(x, node_depth, edge_index, edge_attr, edge_masks, batch, type_emb, attr_emb, depth_emb, edge_W, edge_b, attn_W, attn_b, Wih, Whh, bih, bhh):
    raise NotImplementedError("write your pallas kernel here")


# speedup vs baseline: 1.8072x; 1.8072x over previous
import functools
import jax
import jax.numpy as jnp
from jax import lax
from jax.experimental import pallas as pl
from jax.experimental.pallas import tpu as pltpu
from jax.experimental.pallas import tpu_sc as plsc

_N = 10000
_E = 320000
_D = 128
_NLAYERS = 2
_NLEVELS = 8
_MAXD = 20
_NGRAPHS = 16

_NW = 32
_NLOC = 320
_NP = _NW * _NLOC
_EK = 256
_EP = _E + 2 * _EK
_ER = _EP // 128

_BR = 2048
_GB = _NP // _BR


def _coef_body(sc_ref, ea0_ref, ea1_ref, mf_ref, out_ref):
    l = pl.program_id(0)
    s0 = sc_ref[l, 0]
    s1 = sc_ref[l, 1]
    out_ref[0, 0] = mf_ref[0] * jnp.exp(ea0_ref[...] * s0 + ea1_ref[...] * s1)


def _coef(ew2, ea0, ea1, mf):
    return pl.pallas_call(
        _coef_body,
        grid=(2, 8),
        in_specs=[
            pl.BlockSpec(memory_space=pltpu.SMEM),
            pl.BlockSpec((_ER, 128), lambda l, v: (0, 0)),
            pl.BlockSpec((_ER, 128), lambda l, v: (0, 0)),
            pl.BlockSpec((1, _ER, 128), lambda l, v: (v, 0, 0)),
        ],
        out_specs=pl.BlockSpec((1, 1, _ER, 128), lambda l, v: (l, v, 0, 0)),
        out_shape=jax.ShapeDtypeStruct((2, 8, _ER, 128), jnp.float32),
    )(ew2, ea0, ea1, mf)


def _prep_body(hp_ref, wt_ref, bih_ref, bhh_ref, w2_ref, deg_ref,
               gi_ref, h0_ref, q0_ref):
    gi = jnp.dot(hp_ref[...], wt_ref[...],
                 preferred_element_type=jnp.float32) + bih_ref[...]
    gi_ref[...] = gi
    b_r = bhh_ref[0, 0:_D]
    b_z = bhh_ref[0, _D:2 * _D]
    b_n = bhh_ref[0, 2 * _D:3 * _D]
    r0 = jax.nn.sigmoid(gi[:, 0:_D] + b_r)
    z0 = jax.nn.sigmoid(gi[:, _D:2 * _D] + b_z)
    n0 = jnp.tanh(gi[:, 2 * _D:3 * _D] + r0 * b_n)
    h_root = (1.0 - z0) * n0
    isroot = deg_ref[...] == 0
    h0 = jnp.where(isroot, h_root, 0.0)
    h0_ref[...] = h0
    q0_ref[...] = jnp.dot(h0, w2_ref[...], preferred_element_type=jnp.float32)


def _prep(h_prev, wih_t, bih, bhh, w2, deg2d):
    return pl.pallas_call(
        _prep_body,
        grid=(_GB,),
        in_specs=[
            pl.BlockSpec((_BR, _D), lambda i: (i, 0)),
            pl.BlockSpec((_D, 3 * _D), lambda i: (0, 0)),
            pl.BlockSpec((1, 3 * _D), lambda i: (0, 0)),
            pl.BlockSpec((1, 3 * _D), lambda i: (0, 0)),
            pl.BlockSpec((_D, 1), lambda i: (0, 0)),
            pl.BlockSpec((_BR, 1), lambda i: (i, 0)),
        ],
        out_specs=[
            pl.BlockSpec((_BR, 3 * _D), lambda i: (i, 0)),
            pl.BlockSpec((_BR, _D), lambda i: (i, 0)),
            pl.BlockSpec((_BR, 1), lambda i: (i, 0)),
        ],
        out_shape=[
            jax.ShapeDtypeStruct((_NP, 3 * _D), jnp.float32),
            jax.ShapeDtypeStruct((_NP, _D), jnp.float32),
            jax.ShapeDtypeStruct((_NP, 1), jnp.float32),
        ],
    )(h_prev, wih_t, bih, bhh, w2, deg2d)


def _gru_body(gi_ref, h_ref, hp_ref, mr_ref, ss_ref, wt_ref, bhh_ref, w2_ref,
              hn_ref, qn_ref):
    ss = ss_ref[...]
    nm = ss > 0.0
    recip = jnp.where(nm, 1.0 / jnp.where(nm, ss, 1.0), 0.0)
    msg = mr_ref[...] * recip
    gh = jnp.dot(msg, wt_ref[...],
                 preferred_element_type=jnp.float32) + bhh_ref[...]
    gi = gi_ref[...]
    r = jax.nn.sigmoid(gi[:, 0:_D] + gh[:, 0:_D])
    z = jax.nn.sigmoid(gi[:, _D:2 * _D] + gh[:, _D:2 * _D])
    n = jnp.tanh(gi[:, 2 * _D:3 * _D] + r * gh[:, 2 * _D:3 * _D])
    h_upd = (1.0 - z) * n + z * msg
    hn = jnp.where(nm, h_upd, h_ref[...])
    hn_ref[...] = hn
    qn_ref[...] = jnp.dot(hn, w2_ref[...], preferred_element_type=jnp.float32)


def _gru(gi, h, h_prev, msg_raw, ssum2d, whh_t, bhh, w2):
    return pl.pallas_call(
        _gru_body,
        grid=(_GB,),
        in_specs=[
            pl.BlockSpec((_BR, 3 * _D), lambda i: (i, 0)),
            pl.BlockSpec((_BR, _D), lambda i: (i, 0)),
            pl.BlockSpec((_BR, _D), lambda i: (i, 0)),
            pl.BlockSpec((_BR, _D), lambda i: (i, 0)),
            pl.BlockSpec((_BR, 1), lambda i: (i, 0)),
            pl.BlockSpec((_D, 3 * _D), lambda i: (0, 0)),
            pl.BlockSpec((1, 3 * _D), lambda i: (0, 0)),
            pl.BlockSpec((_D, 1), lambda i: (0, 0)),
        ],
        out_specs=[
            pl.BlockSpec((_BR, _D), lambda i: (i, 0)),
            pl.BlockSpec((_BR, 1), lambda i: (i, 0)),
        ],
        out_shape=[
            jax.ShapeDtypeStruct((_NP, _D), jnp.float32),
            jax.ShapeDtypeStruct((_NP, 1), jnp.float32),
        ],
    )(gi, h, h_prev, msg_raw, ssum2d, whh_t, bhh, w2)


def _pool_body(H_ref, b_ref, lc0_ref, lc1_ref, out_ref):
    @pl.when(pl.program_id(0) == 0)
    def _init():
        out_ref[...] = jnp.full((_NGRAPHS, 2 * _D), -jnp.inf, jnp.float32)

    isleaf = (lc0_ref[...] + lc1_ref[...]) == 0.0
    rows = jnp.where(isleaf, H_ref[...], -1e30)
    b = b_ref[...]
    for g in range(_NGRAPHS):
        m = b == g
        colmax = jnp.max(jnp.where(m, rows, -jnp.inf), axis=0)
        out_ref[g, :] = jnp.maximum(out_ref[g, :], colmax)


def _pool(H, batch2d, lc0, lc1):
    return pl.pallas_call(
        _pool_body,
        grid=(_GB,),
        in_specs=[
            pl.BlockSpec((_BR, 2 * _D), lambda i: (i, 0)),
            pl.BlockSpec((_BR, 1), lambda i: (i, 0)),
            pl.BlockSpec((_BR, 1), lambda i: (i, 0)),
            pl.BlockSpec((_BR, 1), lambda i: (i, 0)),
        ],
        out_specs=pl.BlockSpec((_NGRAPHS, 2 * _D), lambda i: (0, 0)),
        out_shape=jax.ShapeDtypeStruct((_NGRAPHS, 2 * _D), jnp.float32),
    )(H, batch2d, lc0, lc1)


def kernel(x, node_depth, edge_index, edge_attr, edge_masks, batch,
           type_emb, attr_emb, depth_emb, edge_W, edge_b, attn_W, attn_b,
           Wih, Whh, bih, bhh):
    src, dst = edge_index[0], edge_index[1]

    order = jnp.argsort(dst)
    src_s = src[order].astype(jnp.int32)
    dst_s = dst[order].astype(jnp.int32)
    ea_s = edge_attr[order]
    mf_s = edge_masks[:, order].astype(jnp.float32)
    row_ptr = jnp.searchsorted(dst_s, jnp.arange(_NP + 1, dtype=jnp.int32),
                               ).astype(jnp.int32)
    deg2d = (row_ptr[1:] - row_ptr[:-1]).reshape(_NP, 1)
    eb = row_ptr[jnp.arange(_NW + 1) * _NLOC]
    eb = jnp.pad(eb, (0, 7)).astype(jnp.int32)

    padE = _EP - _E
    src_p = jnp.pad(src_s, (0, padE))
    dst_p = jnp.pad(dst_s, (0, padE), constant_values=_NP - 1)
    ea0 = jnp.pad(ea_s[:, 0], (0, padE)).reshape(_ER, 128)
    ea1 = jnp.pad(ea_s[:, 1], (0, padE)).reshape(_ER, 128)
    mfp = jnp.pad(mf_s, ((0, 0), (0, padE))).reshape(8, _ER, 128)

    depth = jnp.minimum(node_depth, _MAXD)
    h_prev = type_emb[x[:, 0]] + attr_emb[x[:, 1]] + depth_emb[depth]
    h_prev = jnp.pad(h_prev, ((0, _NP - _N), (0, 0)))
    lcnt = jnp.zeros((_NP,), jnp.float32).at[src].add(1.0)
    lc0 = lcnt.reshape(_NP, 1)
    lc1 = jnp.zeros((_NP, 1), jnp.float32)

    w2s = attn_W[:, _D:, 0]
    ew2 = jnp.einsum('lcd,ld->lc', edge_W, w2s)
    g_all = _coef(ew2, ea0, ea1, mfp)

    batch2d = jnp.pad(batch, (0, _NP - _N),
                      constant_values=_NGRAPHS).astype(jnp.int32).reshape(_NP, 1)

    h_layers = []
    for l in range(_NLAYERS):
        w2 = attn_W[l, _D:, :]
        wih_t = Wih[l].T
        whh_t = Whh[l].T
        bih2 = bih[l].reshape(1, 3 * _D)
        bhh2 = bhh[l].reshape(1, 3 * _D)
        gi, h, q = _prep(h_prev, wih_t, bih2, bhh2, w2, deg2d)
        for lev in range(_NLEVELS):
            qe = q[:, 0][src_p[:_E]]
            e = g_all[l, lev].reshape(_EP)[:_E] * jnp.exp(qe)
            ssum = jax.ops.segment_sum(e, dst_s, num_segments=_NP)
            msg_raw = jax.ops.segment_sum(e[:, None] * h[src_p[:_E]],
                                          dst_s, num_segments=_NP)
            h, q = _gru(gi, h, h_prev, msg_raw, ssum.reshape(_NP, 1),
                        whh_t, bhh2, w2)
        h_layers.append(h)
        h_prev = h

    H = jnp.concatenate(h_layers, axis=1)
    return _pool(H, batch2d, lc0, lc1)


# speedup vs baseline: 5.8123x; 3.2162x over previous
import functools
import jax
import jax.numpy as jnp
from jax import lax
from jax.experimental import pallas as pl
from jax.experimental.pallas import tpu as pltpu
from jax.experimental.pallas import tpu_sc as plsc

_N = 10000
_E = 320000
_D = 128
_NLAYERS = 2
_NLEVELS = 8
_MAXD = 20
_NGRAPHS = 16

_NW = 32
_NLOC = 320
_NP = _NW * _NLOC
_EK = 128
_EP = _E + 2 * _EK
_ER = _EP // 128

_BR = 2048
_GB = _NP // _BR


def _coef_body(sc_ref, ea0_ref, ea1_ref, mf_ref, out_ref):
    l = pl.program_id(0)
    s0 = sc_ref[l, 0]
    s1 = sc_ref[l, 1]
    out_ref[0, 0] = mf_ref[0] * jnp.exp(ea0_ref[...] * s0 + ea1_ref[...] * s1)


def _coef(ew2, ea0, ea1, mf):
    return pl.pallas_call(
        _coef_body,
        grid=(2, 8),
        in_specs=[
            pl.BlockSpec(memory_space=pltpu.SMEM),
            pl.BlockSpec((_ER, 128), lambda l, v: (0, 0)),
            pl.BlockSpec((_ER, 128), lambda l, v: (0, 0)),
            pl.BlockSpec((1, _ER, 128), lambda l, v: (v, 0, 0)),
        ],
        out_specs=pl.BlockSpec((1, 1, _ER, 128), lambda l, v: (l, v, 0, 0)),
        out_shape=jax.ShapeDtypeStruct((2, 8, _ER, 128), jnp.float32),
    )(ew2, ea0, ea1, mf)


def _prep_body(hp_ref, wt_ref, bih_ref, bhh_ref, w2_ref, deg_ref,
               gi_ref, h0_ref, q0_ref):
    gi = jnp.dot(hp_ref[...], wt_ref[...],
                 preferred_element_type=jnp.float32) + bih_ref[...]
    gi_ref[...] = gi
    b_r = bhh_ref[0, 0:_D]
    b_z = bhh_ref[0, _D:2 * _D]
    b_n = bhh_ref[0, 2 * _D:3 * _D]
    r0 = jax.nn.sigmoid(gi[:, 0:_D] + b_r)
    z0 = jax.nn.sigmoid(gi[:, _D:2 * _D] + b_z)
    n0 = jnp.tanh(gi[:, 2 * _D:3 * _D] + r0 * b_n)
    h_root = (1.0 - z0) * n0
    isroot = deg_ref[...] == 0
    h0 = jnp.where(isroot, h_root, 0.0)
    h0_ref[...] = h0
    q0_ref[...] = jnp.dot(h0, w2_ref[...], preferred_element_type=jnp.float32)


def _prep(h_prev, wih_t, bih, bhh, w2, deg2d):
    return pl.pallas_call(
        _prep_body,
        grid=(_GB,),
        in_specs=[
            pl.BlockSpec((_BR, _D), lambda i: (i, 0)),
            pl.BlockSpec((_D, 3 * _D), lambda i: (0, 0)),
            pl.BlockSpec((1, 3 * _D), lambda i: (0, 0)),
            pl.BlockSpec((1, 3 * _D), lambda i: (0, 0)),
            pl.BlockSpec((_D, 1), lambda i: (0, 0)),
            pl.BlockSpec((_BR, 1), lambda i: (i, 0)),
        ],
        out_specs=[
            pl.BlockSpec((_BR, 3 * _D), lambda i: (i, 0)),
            pl.BlockSpec((_BR, _D), lambda i: (i, 0)),
            pl.BlockSpec((_BR, 1), lambda i: (i, 0)),
        ],
        out_shape=[
            jax.ShapeDtypeStruct((_NP, 3 * _D), jnp.float32),
            jax.ShapeDtypeStruct((_NP, _D), jnp.float32),
            jax.ShapeDtypeStruct((_NP, 1), jnp.float32),
        ],
    )(h_prev, wih_t, bih, bhh, w2, deg2d)


def _gru_body(gi_ref, h_ref, hp_ref, mr_ref, ss_ref, wt_ref, bhh_ref, w2_ref,
              hn_ref, qn_ref):
    ss = ss_ref[...]
    nm = ss > 0.0
    recip = jnp.where(nm, 1.0 / jnp.where(nm, ss, 1.0), 0.0)
    msg = mr_ref[...] * recip
    gh = jnp.dot(msg, wt_ref[...],
                 preferred_element_type=jnp.float32) + bhh_ref[...]
    gi = gi_ref[...]
    r = jax.nn.sigmoid(gi[:, 0:_D] + gh[:, 0:_D])
    z = jax.nn.sigmoid(gi[:, _D:2 * _D] + gh[:, _D:2 * _D])
    n = jnp.tanh(gi[:, 2 * _D:3 * _D] + r * gh[:, 2 * _D:3 * _D])
    h_upd = (1.0 - z) * n + z * msg
    hn = jnp.where(nm, h_upd, h_ref[...])
    hn_ref[...] = hn
    qn_ref[...] = jnp.dot(hn, w2_ref[...], preferred_element_type=jnp.float32)


def _gru(gi, h, h_prev, msg_raw, ssum2d, whh_t, bhh, w2):
    return pl.pallas_call(
        _gru_body,
        grid=(_GB,),
        in_specs=[
            pl.BlockSpec((_BR, 3 * _D), lambda i: (i, 0)),
            pl.BlockSpec((_BR, _D), lambda i: (i, 0)),
            pl.BlockSpec((_BR, _D), lambda i: (i, 0)),
            pl.BlockSpec((_BR, _D), lambda i: (i, 0)),
            pl.BlockSpec((_BR, 1), lambda i: (i, 0)),
            pl.BlockSpec((_D, 3 * _D), lambda i: (0, 0)),
            pl.BlockSpec((1, 3 * _D), lambda i: (0, 0)),
            pl.BlockSpec((_D, 1), lambda i: (0, 0)),
        ],
        out_specs=[
            pl.BlockSpec((_BR, _D), lambda i: (i, 0)),
            pl.BlockSpec((_BR, 1), lambda i: (i, 0)),
        ],
        out_shape=[
            jax.ShapeDtypeStruct((_NP, _D), jnp.float32),
            jax.ShapeDtypeStruct((_NP, 1), jnp.float32),
        ],
    )(gi, h, h_prev, msg_raw, ssum2d, whh_t, bhh, w2)


def _pool_body(H_ref, b_ref, lc0_ref, lc1_ref, out_ref):
    @pl.when(pl.program_id(0) == 0)
    def _init():
        out_ref[...] = jnp.full((_NGRAPHS, 2 * _D), -jnp.inf, jnp.float32)

    isleaf = (lc0_ref[...] + lc1_ref[...]) == 0.0
    rows = jnp.where(isleaf, H_ref[...], -1e30)
    b = b_ref[...]
    for g in range(_NGRAPHS):
        m = b == g
        colmax = jnp.max(jnp.where(m, rows, -jnp.inf), axis=0)
        out_ref[g, :] = jnp.maximum(out_ref[g, :], colmax)


def _pool(H, batch2d, lc0, lc1):
    return pl.pallas_call(
        _pool_body,
        grid=(_GB,),
        in_specs=[
            pl.BlockSpec((_BR, 2 * _D), lambda i: (i, 0)),
            pl.BlockSpec((_BR, 1), lambda i: (i, 0)),
            pl.BlockSpec((_BR, 1), lambda i: (i, 0)),
            pl.BlockSpec((_BR, 1), lambda i: (i, 0)),
        ],
        out_specs=pl.BlockSpec((_NGRAPHS, 2 * _D), lambda i: (0, 0)),
        out_shape=jax.ShapeDtypeStruct((_NGRAPHS, 2 * _D), jnp.float32),
    )(H, batch2d, lc0, lc1)


def _iota16():
    return lax.broadcasted_iota(jnp.int32, (16,), 0)


def _sget(ref, i, nchunks):
    acc = jnp.int32(0)
    for c in range(nchunks):
        v = ref[pl.ds(c * 16, 16)]
        lane = _iota16() + c * 16
        acc = jnp.maximum(acc, jnp.max(jnp.where(lane == i, v, 0)))
    return acc


def _make_level_sc():
    mesh = plsc.VectorSubcoreMesh(core_axis_name="c", subcore_axis_name="s")

    @functools.partial(
        pl.kernel,
        out_type=[
            jax.ShapeDtypeStruct((_NP, _D), jnp.float32),
            jax.ShapeDtypeStruct((_NP,), jnp.float32),
        ],
        mesh=mesh,
        compiler_params=pltpu.CompilerParams(needs_layout_passes=False),
        scratch_types=[
            pltpu.VMEM((_NP,), jnp.float32),
            pltpu.VMEM((_NLOC, _D), jnp.float32),
            pltpu.VMEM((_NLOC,), jnp.float32),
            pltpu.VMEM((_EK,), jnp.int32),
            pltpu.VMEM((_EK,), jnp.int32),
            pltpu.VMEM((_EK,), jnp.float32),
            pltpu.VMEM((_EK,), jnp.float32),
            pltpu.VMEM((_EK,), jnp.int32),
            pltpu.VMEM((_EK, _D), jnp.float32),
            pltpu.VMEM((48,), jnp.int32),
            pltpu.SemaphoreType.DMA,
        ],
    )
    def level_sc(q_hbm, h_hbm, src_hbm, dst_hbm, g_hbm, eb_hbm, z2_hbm,
                 z1_hbm, msg_hbm, ssum_hbm,
                 q_loc, msg_loc, ssum_loc, src_c, dst_c, g_c, e_c, dl_c,
                 rows, ebw, sem):
        wid = lax.axis_index("c") * 16 + lax.axis_index("s")
        node_base = pl.multiple_of(wid * _NLOC, _NLOC)
        pltpu.sync_copy(eb_hbm, ebw)
        pltpu.sync_copy(q_hbm, q_loc)
        pltpu.sync_copy(z2_hbm, msg_loc)
        pltpu.sync_copy(z1_hbm, ssum_loc)
        ebpair = ebw[pl.ds(wid, 16)]
        estart = ebpair[0]
        eend = ebpair[1]
        base0 = jnp.bitwise_and(estart, jnp.int32(-8))
        nch = (eend - base0 + _EK - 1) // _EK

        def chunk_body(ch, carry):
            base = pl.multiple_of(base0 + ch * _EK, 8)
            pltpu.sync_copy(src_hbm.at[pl.ds(base, _EK)], src_c)
            pltpu.sync_copy(dst_hbm.at[pl.ds(base, _EK)], dst_c)
            pltpu.sync_copy(g_hbm.at[pl.ds(base, _EK)], g_c)
            pltpu.async_copy(h_hbm.at[src_c], rows, sem).wait()
            for j in range(_EK // 16):
                sl = pl.ds(j * 16, 16)
                qs = plsc.load_gather(q_loc, [src_c[sl]])
                eidx = base + j * 16 + _iota16()
                ok = (eidx >= estart) & (eidx < eend)
                ev = jnp.where(ok, g_c[sl] * jnp.exp(qs), 0.0)
                dl = jnp.clip(dst_c[sl] - node_base, 0, _NLOC - 1)
                plsc.addupdate_scatter(ssum_loc, [dl], ev)
                e_c[sl] = ev
                dl_c[sl] = dl

            def edge_body(i, c2):
                spl = jnp.zeros((16,), jnp.int32) + i
                ev = plsc.load_gather(e_c, [spl])
                dl = plsc.load_gather(dl_c, [spl])
                for j in range(_D // 16):
                    col = j * 16 + _iota16()
                    rv = plsc.load_gather(rows, [spl, col])
                    plsc.addupdate_scatter(msg_loc, [dl, col], ev * rv)
                return c2

            lax.fori_loop(0, _EK, edge_body, 0)
            return carry

        lax.fori_loop(0, nch, chunk_body, 0)
        pltpu.sync_copy(msg_loc, msg_hbm.at[pl.ds(node_base, _NLOC)])
        pltpu.sync_copy(ssum_loc, ssum_hbm.at[pl.ds(node_base, _NLOC)])

    return level_sc


@functools.lru_cache(maxsize=None)
def _level_sc_cached():
    return _make_level_sc()


def kernel(x, node_depth, edge_index, edge_attr, edge_masks, batch,
           type_emb, attr_emb, depth_emb, edge_W, edge_b, attn_W, attn_b,
           Wih, Whh, bih, bhh):
    src, dst = edge_index[0], edge_index[1]

    order = jnp.argsort(dst)
    src_s = src[order].astype(jnp.int32)
    dst_s = dst[order].astype(jnp.int32)
    ea_s = edge_attr[order]
    mf_s = edge_masks[:, order].astype(jnp.float32)
    row_ptr = jnp.searchsorted(dst_s, jnp.arange(_NP + 1, dtype=jnp.int32),
                               ).astype(jnp.int32)
    deg2d = (row_ptr[1:] - row_ptr[:-1]).reshape(_NP, 1)
    eb = row_ptr[jnp.arange(_NW + 1) * _NLOC]
    eb = jnp.pad(eb, (0, 15)).astype(jnp.int32)

    padE = _EP - _E
    src_p = jnp.pad(src_s, (0, padE))
    dst_p = jnp.pad(dst_s, (0, padE), constant_values=_NP - 1)
    ea0 = jnp.pad(ea_s[:, 0], (0, padE)).reshape(_ER, 128)
    ea1 = jnp.pad(ea_s[:, 1], (0, padE)).reshape(_ER, 128)
    mfp = jnp.pad(mf_s, ((0, 0), (0, padE))).reshape(8, _ER, 128)

    depth = jnp.minimum(node_depth, _MAXD)
    h_prev = type_emb[x[:, 0]] + attr_emb[x[:, 1]] + depth_emb[depth]
    h_prev = jnp.pad(h_prev, ((0, _NP - _N), (0, 0)))
    lcnt = jnp.zeros((_NP,), jnp.float32).at[src].add(1.0)
    lc0 = lcnt.reshape(_NP, 1)
    lc1 = jnp.zeros((_NP, 1), jnp.float32)

    w2s = attn_W[:, _D:, 0]
    ew2 = jnp.einsum('lcd,ld->lc', edge_W, w2s)
    g_all = _coef(ew2, ea0, ea1, mfp)
    g2 = g_all.reshape(2, 8, _EP)
    z2 = jnp.zeros((_NLOC, _D), jnp.float32)
    z1 = jnp.zeros((_NLOC,), jnp.float32)
    level_sc = _level_sc_cached()

    batch2d = jnp.pad(batch, (0, _NP - _N),
                      constant_values=_NGRAPHS).astype(jnp.int32).reshape(_NP, 1)

    h_layers = []
    for l in range(_NLAYERS):
        w2 = attn_W[l, _D:, :]
        wih_t = Wih[l].T
        whh_t = Whh[l].T
        bih2 = bih[l].reshape(1, 3 * _D)
        bhh2 = bhh[l].reshape(1, 3 * _D)
        gi, h, q = _prep(h_prev, wih_t, bih2, bhh2, w2, deg2d)
        for lev in range(_NLEVELS):
            msg_raw, ssum = level_sc(q.reshape(_NP), h, src_p, dst_p,
                                     g2[l, lev], eb, z2, z1)
            h, q = _gru(gi, h, h_prev, msg_raw, ssum.reshape(_NP, 1),
                        whh_t, bhh2, w2)
        h_layers.append(h)
        h_prev = h

    H = jnp.concatenate(h_layers, axis=1)
    return _pool(H, batch2d, lc0, lc1)


# speedup vs baseline: 7.1746x; 1.2344x over previous
import functools
import jax
import jax.numpy as jnp
from jax import lax
from jax.experimental import pallas as pl
from jax.experimental.pallas import tpu as pltpu
from jax.experimental.pallas import tpu_sc as plsc

_N = 10000
_E = 320000
_D = 128
_NLAYERS = 2
_NLEVELS = 8
_MAXD = 20
_NGRAPHS = 16

_NW = 32
_NLOC = 320
_NP = _NW * _NLOC
_EK = 128
_EP = _E + 4 * _EK
_ER = _EP // 128

_BR = 2048
_GB = _NP // _BR


def _coef_body(sc_ref, ea0_ref, ea1_ref, mf_ref, out_ref):
    l = pl.program_id(0)
    s0 = sc_ref[l, 0]
    s1 = sc_ref[l, 1]
    out_ref[0, 0] = mf_ref[0] * jnp.exp(ea0_ref[...] * s0 + ea1_ref[...] * s1)


def _coef(ew2, ea0, ea1, mf):
    return pl.pallas_call(
        _coef_body,
        grid=(2, 8),
        in_specs=[
            pl.BlockSpec(memory_space=pltpu.SMEM),
            pl.BlockSpec((_ER, 128), lambda l, v: (0, 0)),
            pl.BlockSpec((_ER, 128), lambda l, v: (0, 0)),
            pl.BlockSpec((1, _ER, 128), lambda l, v: (v, 0, 0)),
        ],
        out_specs=pl.BlockSpec((1, 1, _ER, 128), lambda l, v: (l, v, 0, 0)),
        out_shape=jax.ShapeDtypeStruct((2, 8, _ER, 128), jnp.float32),
    )(ew2, ea0, ea1, mf)


def _prep_body(hp_ref, wt_ref, bih_ref, bhh_ref, w2_ref, deg_ref,
               gi_ref, h0_ref, q0_ref):
    gi = jnp.dot(hp_ref[...], wt_ref[...],
                 preferred_element_type=jnp.float32) + bih_ref[...]
    gi_ref[...] = gi
    b_r = bhh_ref[0, 0:_D]
    b_z = bhh_ref[0, _D:2 * _D]
    b_n = bhh_ref[0, 2 * _D:3 * _D]
    r0 = jax.nn.sigmoid(gi[:, 0:_D] + b_r)
    z0 = jax.nn.sigmoid(gi[:, _D:2 * _D] + b_z)
    n0 = jnp.tanh(gi[:, 2 * _D:3 * _D] + r0 * b_n)
    h_root = (1.0 - z0) * n0
    isroot = deg_ref[...] == 0
    h0 = jnp.where(isroot, h_root, 0.0)
    h0_ref[...] = h0
    q0_ref[...] = jnp.dot(h0, w2_ref[...], preferred_element_type=jnp.float32)


def _prep(h_prev, wih_t, bih, bhh, w2, deg2d):
    return pl.pallas_call(
        _prep_body,
        grid=(_GB,),
        in_specs=[
            pl.BlockSpec((_BR, _D), lambda i: (i, 0)),
            pl.BlockSpec((_D, 3 * _D), lambda i: (0, 0)),
            pl.BlockSpec((1, 3 * _D), lambda i: (0, 0)),
            pl.BlockSpec((1, 3 * _D), lambda i: (0, 0)),
            pl.BlockSpec((_D, 1), lambda i: (0, 0)),
            pl.BlockSpec((_BR, 1), lambda i: (i, 0)),
        ],
        out_specs=[
            pl.BlockSpec((_BR, 3 * _D), lambda i: (i, 0)),
            pl.BlockSpec((_BR, _D), lambda i: (i, 0)),
            pl.BlockSpec((_BR, 1), lambda i: (i, 0)),
        ],
        out_shape=[
            jax.ShapeDtypeStruct((_NP, 3 * _D), jnp.float32),
            jax.ShapeDtypeStruct((_NP, _D), jnp.float32),
            jax.ShapeDtypeStruct((_NP, 1), jnp.float32),
        ],
    )(h_prev, wih_t, bih, bhh, w2, deg2d)


def _gru_body(gi_ref, h_ref, hp_ref, mr_ref, ss_ref, wt_ref, bhh_ref, w2_ref,
              hn_ref, qn_ref):
    ss = ss_ref[...]
    nm = ss > 0.0
    recip = jnp.where(nm, 1.0 / jnp.where(nm, ss, 1.0), 0.0)
    msg = mr_ref[...] * recip
    gh = jnp.dot(msg, wt_ref[...],
                 preferred_element_type=jnp.float32) + bhh_ref[...]
    gi = gi_ref[...]
    r = jax.nn.sigmoid(gi[:, 0:_D] + gh[:, 0:_D])
    z = jax.nn.sigmoid(gi[:, _D:2 * _D] + gh[:, _D:2 * _D])
    n = jnp.tanh(gi[:, 2 * _D:3 * _D] + r * gh[:, 2 * _D:3 * _D])
    h_upd = (1.0 - z) * n + z * msg
    hn = jnp.where(nm, h_upd, h_ref[...])
    hn_ref[...] = hn
    qn_ref[...] = jnp.dot(hn, w2_ref[...], preferred_element_type=jnp.float32)


def _gru(gi, h, h_prev, msg_raw, ssum2d, whh_t, bhh, w2):
    return pl.pallas_call(
        _gru_body,
        grid=(_GB,),
        in_specs=[
            pl.BlockSpec((_BR, 3 * _D), lambda i: (i, 0)),
            pl.BlockSpec((_BR, _D), lambda i: (i, 0)),
            pl.BlockSpec((_BR, _D), lambda i: (i, 0)),
            pl.BlockSpec((_BR, _D), lambda i: (i, 0)),
            pl.BlockSpec((_BR, 1), lambda i: (i, 0)),
            pl.BlockSpec((_D, 3 * _D), lambda i: (0, 0)),
            pl.BlockSpec((1, 3 * _D), lambda i: (0, 0)),
            pl.BlockSpec((_D, 1), lambda i: (0, 0)),
        ],
        out_specs=[
            pl.BlockSpec((_BR, _D), lambda i: (i, 0)),
            pl.BlockSpec((_BR, 1), lambda i: (i, 0)),
        ],
        out_shape=[
            jax.ShapeDtypeStruct((_NP, _D), jnp.float32),
            jax.ShapeDtypeStruct((_NP, 1), jnp.float32),
        ],
    )(gi, h, h_prev, msg_raw, ssum2d, whh_t, bhh, w2)


def _pool_body(H_ref, b_ref, lc0_ref, lc1_ref, out_ref):
    @pl.when(pl.program_id(0) == 0)
    def _init():
        out_ref[...] = jnp.full((_NGRAPHS, 2 * _D), -jnp.inf, jnp.float32)

    isleaf = (lc0_ref[...] + lc1_ref[...]) == 0.0
    rows = jnp.where(isleaf, H_ref[...], -1e30)
    b = b_ref[...]
    for g in range(_NGRAPHS):
        m = b == g
        colmax = jnp.max(jnp.where(m, rows, -jnp.inf), axis=0)
        out_ref[g, :] = jnp.maximum(out_ref[g, :], colmax)


def _pool(H, batch2d, lc0, lc1):
    return pl.pallas_call(
        _pool_body,
        grid=(_GB,),
        in_specs=[
            pl.BlockSpec((_BR, 2 * _D), lambda i: (i, 0)),
            pl.BlockSpec((_BR, 1), lambda i: (i, 0)),
            pl.BlockSpec((_BR, 1), lambda i: (i, 0)),
            pl.BlockSpec((_BR, 1), lambda i: (i, 0)),
        ],
        out_specs=pl.BlockSpec((_NGRAPHS, 2 * _D), lambda i: (0, 0)),
        out_shape=jax.ShapeDtypeStruct((_NGRAPHS, 2 * _D), jnp.float32),
    )(H, batch2d, lc0, lc1)


def _iota16():
    return lax.broadcasted_iota(jnp.int32, (16,), 0)


def _sget(ref, i, nchunks):
    acc = jnp.int32(0)
    for c in range(nchunks):
        v = ref[pl.ds(c * 16, 16)]
        lane = _iota16() + c * 16
        acc = jnp.maximum(acc, jnp.max(jnp.where(lane == i, v, 0)))
    return acc


def _make_level_sc():
    mesh = plsc.VectorSubcoreMesh(core_axis_name="c", subcore_axis_name="s")

    @functools.partial(
        pl.kernel,
        out_type=[
            jax.ShapeDtypeStruct((_NP, _D), jnp.float32),
            jax.ShapeDtypeStruct((_NP,), jnp.float32),
        ],
        mesh=mesh,
        compiler_params=pltpu.CompilerParams(needs_layout_passes=False),
        scratch_types=[
            pltpu.VMEM((_NP,), jnp.float32),
            pltpu.VMEM((_NLOC, _D), jnp.float32),
            pltpu.VMEM((_NLOC,), jnp.float32),
            pltpu.VMEM((2 * _EK,), jnp.int32),
            pltpu.VMEM((2 * _EK,), jnp.int32),
            pltpu.VMEM((2 * _EK,), jnp.float32),
            pltpu.VMEM((2 * _EK + 16,), jnp.float32),
            pltpu.VMEM((2 * _EK, _D), jnp.float32),
            pltpu.VMEM((48,), jnp.int32),
            pltpu.SemaphoreType.DMA,
            pltpu.SemaphoreType.DMA,
            pltpu.SemaphoreType.DMA,
            pltpu.SemaphoreType.DMA,
        ],
    )
    def level_sc(q_hbm, h_hbm, src_hbm, dst_hbm, g_hbm, eb_hbm, z2_hbm,
                 z1_hbm, msg_hbm, ssum_hbm,
                 q_loc, msg_loc, ssum_loc, src_c, dst_c, g_c, ed_c,
                 rows, ebw, lsem0, lsem1, gsem0, gsem1):
        wid = lax.axis_index("c") * 16 + lax.axis_index("s")
        node_base = pl.multiple_of(wid * _NLOC, _NLOC)
        pltpu.sync_copy(eb_hbm, ebw)
        pltpu.sync_copy(q_hbm, q_loc)
        pltpu.sync_copy(z2_hbm, msg_loc)
        pltpu.sync_copy(z1_hbm, ssum_loc)
        ebpair = ebw[pl.ds(wid, 16)]
        estart = ebpair[0]
        eend = ebpair[1]
        base0 = jnp.bitwise_and(estart, jnp.int32(-8))
        nch = (eend - base0 + _EK - 1) // _EK
        lsems = (lsem0, lsem1)
        gsems = (gsem0, gsem1)

        def issue_linear(ch, p):
            base = pl.multiple_of(base0 + ch * _EK, 8)
            off = p * _EK
            pltpu.async_copy(src_hbm.at[pl.ds(base, _EK)],
                             src_c.at[pl.ds(off, _EK)], lsems[p])
            pltpu.async_copy(dst_hbm.at[pl.ds(base, _EK)],
                             dst_c.at[pl.ds(off, _EK)], lsems[p])
            pltpu.async_copy(g_hbm.at[pl.ds(base, _EK)],
                             g_c.at[pl.ds(off, _EK)], lsems[p])

        def wait_linear(p):
            base = pl.multiple_of(base0, 8)
            off = p * _EK
            pltpu.make_async_copy(src_hbm.at[pl.ds(base, _EK)],
                                  src_c.at[pl.ds(off, _EK)], lsems[p]).wait()
            pltpu.make_async_copy(dst_hbm.at[pl.ds(base, _EK)],
                                  dst_c.at[pl.ds(off, _EK)], lsems[p]).wait()
            pltpu.make_async_copy(g_hbm.at[pl.ds(base, _EK)],
                                  g_c.at[pl.ds(off, _EK)], lsems[p]).wait()

        def issue_gather(p):
            off = p * _EK
            pltpu.async_copy(h_hbm.at[src_c.at[pl.ds(off, _EK)]],
                             rows.at[pl.ds(off, _EK)], gsems[p])

        def wait_gather(p):
            off = p * _EK
            pltpu.make_async_copy(h_hbm.at[src_c.at[pl.ds(off, _EK)]],
                                  rows.at[pl.ds(off, _EK)], gsems[p]).wait()

        _PIPE = False
        if _PIPE:
            issue_linear(0, 0)
            issue_linear(1, 1)
            wait_linear(0)
            issue_gather(0)

        def half_chunk(ch, p, carry):
            base = pl.multiple_of(base0 + ch * _EK, 8)
            off = p * _EK
            if _PIPE:
                wait_gather(p)
                wait_linear(1 - p)
                issue_gather(1 - p)
            else:
                pltpu.sync_copy(src_hbm.at[pl.ds(base, _EK)],
                                src_c.at[pl.ds(off, _EK)])
                pltpu.sync_copy(dst_hbm.at[pl.ds(base, _EK)],
                                dst_c.at[pl.ds(off, _EK)])
                pltpu.sync_copy(g_hbm.at[pl.ds(base, _EK)],
                                g_c.at[pl.ds(off, _EK)])
                pltpu.async_copy(h_hbm.at[src_c.at[pl.ds(off, _EK)]],
                                 rows.at[pl.ds(off, _EK)], gsems[p]).wait()
            for j in range(_EK // 16):
                sl = pl.ds(off + j * 16, 16)
                srcv = src_c[sl]
                qs = plsc.load_gather(q_loc, [srcv])
                eidx = base + j * 16 + _iota16()
                ok = (eidx >= estart) & (eidx < eend)
                ev = jnp.where(ok, g_c[sl] * jnp.exp(qs), 0.0)
                dl = jnp.clip(dst_c[sl] - node_base, 0, _NLOC - 1)
                plsc.addupdate_scatter(ssum_loc, [dl], ev)
                pos = 2 * (j * 16) + 2 * _iota16()
                plsc.store_scatter(ed_c, [pos], ev)
                plsc.store_scatter(ed_c, [pos + 1], dl.astype(jnp.float32))

            def edge_body(i, c2):
                cur_, b0, b1, b2, b3, b4, b5, b6, b7 = c2
                ed = ed_c[pl.ds(2 * i, 16)]
                ev_s = ed[0]
                dl_s = ed[1]

                def flush(cc):
                    cu = cc[0]
                    ci = cu.astype(jnp.int32)
                    msg_loc[ci, pl.ds(0, 16)] = cc[1]
                    msg_loc[ci, pl.ds(16, 16)] = cc[2]
                    msg_loc[ci, pl.ds(32, 16)] = cc[3]
                    msg_loc[ci, pl.ds(48, 16)] = cc[4]
                    msg_loc[ci, pl.ds(64, 16)] = cc[5]
                    msg_loc[ci, pl.ds(80, 16)] = cc[6]
                    msg_loc[ci, pl.ds(96, 16)] = cc[7]
                    msg_loc[ci, pl.ds(112, 16)] = cc[8]
                    z = jnp.zeros((16,), jnp.float32)
                    return (dl_s, z, z, z, z, z, z, z, z)

                cur_, b0, b1, b2, b3, b4, b5, b6, b7 = lax.cond(
                    dl_s != cur_, flush, lambda cc: cc,
                    (cur_, b0, b1, b2, b3, b4, b5, b6, b7))
                evb = jnp.full((16,), ev_s)
                ri = off + i
                b0 = b0 + evb * rows[ri, pl.ds(0, 16)]
                b1 = b1 + evb * rows[ri, pl.ds(16, 16)]
                b2 = b2 + evb * rows[ri, pl.ds(32, 16)]
                b3 = b3 + evb * rows[ri, pl.ds(48, 16)]
                b4 = b4 + evb * rows[ri, pl.ds(64, 16)]
                b5 = b5 + evb * rows[ri, pl.ds(80, 16)]
                b6 = b6 + evb * rows[ri, pl.ds(96, 16)]
                b7 = b7 + evb * rows[ri, pl.ds(112, 16)]
                return (cur_, b0, b1, b2, b3, b4, b5, b6, b7)

            carry = lax.fori_loop(0, _EK, edge_body, carry)
            if _PIPE:
                issue_linear(ch + 2, p)
            return carry

        def pair_body(cp, carry):
            carry = half_chunk(2 * cp, 0, carry)
            carry = half_chunk(2 * cp + 1, 1, carry)
            return carry

        npairs = (nch + 1) // 2
        z16 = jnp.zeros((16,), jnp.float32)
        fin = lax.fori_loop(0, npairs, pair_body,
                            (jnp.float32(0.0), z16, z16, z16, z16, z16, z16,
                             z16, z16))
        ci = fin[0].astype(jnp.int32)
        msg_loc[ci, pl.ds(0, 16)] = fin[1]
        msg_loc[ci, pl.ds(16, 16)] = fin[2]
        msg_loc[ci, pl.ds(32, 16)] = fin[3]
        msg_loc[ci, pl.ds(48, 16)] = fin[4]
        msg_loc[ci, pl.ds(64, 16)] = fin[5]
        msg_loc[ci, pl.ds(80, 16)] = fin[6]
        msg_loc[ci, pl.ds(96, 16)] = fin[7]
        msg_loc[ci, pl.ds(112, 16)] = fin[8]
        if _PIPE:
            wait_gather(0)
            wait_linear(1)
        pltpu.sync_copy(msg_loc, msg_hbm.at[pl.ds(node_base, _NLOC)])
        pltpu.sync_copy(ssum_loc, ssum_hbm.at[pl.ds(node_base, _NLOC)])

    return level_sc


@functools.lru_cache(maxsize=None)
def _level_sc_cached():
    return _make_level_sc()


def kernel(x, node_depth, edge_index, edge_attr, edge_masks, batch,
           type_emb, attr_emb, depth_emb, edge_W, edge_b, attn_W, attn_b,
           Wih, Whh, bih, bhh):
    src, dst = edge_index[0], edge_index[1]

    order = jnp.argsort(dst)
    src_s = src[order].astype(jnp.int32)
    dst_s = dst[order].astype(jnp.int32)
    ea_s = edge_attr[order]
    mf_s = edge_masks[:, order].astype(jnp.float32)
    row_ptr = jnp.searchsorted(dst_s, jnp.arange(_NP + 1, dtype=jnp.int32),
                               ).astype(jnp.int32)
    deg2d = (row_ptr[1:] - row_ptr[:-1]).reshape(_NP, 1)
    eb = row_ptr[jnp.arange(_NW + 1) * _NLOC]
    eb = jnp.pad(eb, (0, 15)).astype(jnp.int32)

    padE = _EP - _E
    src_p = jnp.pad(src_s, (0, padE))
    dst_p = jnp.pad(dst_s, (0, padE), constant_values=_NP - 1)
    ea0 = jnp.pad(ea_s[:, 0], (0, padE)).reshape(_ER, 128)
    ea1 = jnp.pad(ea_s[:, 1], (0, padE)).reshape(_ER, 128)
    mfp = jnp.pad(mf_s, ((0, 0), (0, padE))).reshape(8, _ER, 128)

    depth = jnp.minimum(node_depth, _MAXD)
    h_prev = type_emb[x[:, 0]] + attr_emb[x[:, 1]] + depth_emb[depth]
    h_prev = jnp.pad(h_prev, ((0, _NP - _N), (0, 0)))
    lcnt = jnp.zeros((_NP,), jnp.float32).at[src].add(1.0)
    lc0 = lcnt.reshape(_NP, 1)
    lc1 = jnp.zeros((_NP, 1), jnp.float32)

    w2s = attn_W[:, _D:, 0]
    ew2 = jnp.einsum('lcd,ld->lc', edge_W, w2s)
    g_all = _coef(ew2, ea0, ea1, mfp)
    g2 = g_all.reshape(2, 8, _EP)
    z2 = jnp.zeros((_NLOC, _D), jnp.float32)
    z1 = jnp.zeros((_NLOC,), jnp.float32)
    level_sc = _level_sc_cached()

    batch2d = jnp.pad(batch, (0, _NP - _N),
                      constant_values=_NGRAPHS).astype(jnp.int32).reshape(_NP, 1)

    h_layers = []
    for l in range(_NLAYERS):
        w2 = attn_W[l, _D:, :]
        wih_t = Wih[l].T
        whh_t = Whh[l].T
        bih2 = bih[l].reshape(1, 3 * _D)
        bhh2 = bhh[l].reshape(1, 3 * _D)
        gi, h, q = _prep(h_prev, wih_t, bih2, bhh2, w2, deg2d)
        for lev in range(_NLEVELS):
            msg_raw, ssum = level_sc(q.reshape(_NP), h, src_p, dst_p,
                                     g2[l, lev], eb, z2, z1)
            h, q = _gru(gi, h, h_prev, msg_raw, ssum.reshape(_NP, 1),
                        whh_t, bhh2, w2)
        h_layers.append(h)
        h_prev = h

    H = jnp.concatenate(h_layers, axis=1)
    return _pool(H, batch2d, lc0, lc1)


# speedup vs baseline: 8.0138x; 1.1170x over previous
import functools
import jax
import jax.numpy as jnp
from jax import lax
from jax.experimental import pallas as pl
from jax.experimental.pallas import tpu as pltpu
from jax.experimental.pallas import tpu_sc as plsc

_N = 10000
_E = 320000
_D = 128
_NLAYERS = 2
_NLEVELS = 8
_MAXD = 20
_NGRAPHS = 16

_NW = 32
_NLOC = 320
_NP = _NW * _NLOC
_EK = 128
_EP = _E + 4 * _EK
_ER = _EP // 128

_BR = 2048
_GB = _NP // _BR


def _coef_body(sc_ref, ea0_ref, ea1_ref, mf_ref, out_ref):
    l = pl.program_id(0)
    s0 = sc_ref[l, 0]
    s1 = sc_ref[l, 1]
    out_ref[0, 0] = mf_ref[0] * jnp.exp(ea0_ref[...] * s0 + ea1_ref[...] * s1)


def _coef(ew2, ea0, ea1, mf):
    return pl.pallas_call(
        _coef_body,
        grid=(2, 8),
        in_specs=[
            pl.BlockSpec(memory_space=pltpu.SMEM),
            pl.BlockSpec((_ER, 128), lambda l, v: (0, 0)),
            pl.BlockSpec((_ER, 128), lambda l, v: (0, 0)),
            pl.BlockSpec((1, _ER, 128), lambda l, v: (v, 0, 0)),
        ],
        out_specs=pl.BlockSpec((1, 1, _ER, 128), lambda l, v: (l, v, 0, 0)),
        out_shape=jax.ShapeDtypeStruct((2, 8, _ER, 128), jnp.float32),
    )(ew2, ea0, ea1, mf)


def _prep_body(hp_ref, wt_ref, bih_ref, bhh_ref, w2_ref, deg_ref,
               gi_ref, h0_ref, q0_ref):
    gi = jnp.dot(hp_ref[...], wt_ref[...],
                 preferred_element_type=jnp.float32) + bih_ref[...]
    gi_ref[...] = gi
    b_r = bhh_ref[0, 0:_D]
    b_z = bhh_ref[0, _D:2 * _D]
    b_n = bhh_ref[0, 2 * _D:3 * _D]
    r0 = jax.nn.sigmoid(gi[:, 0:_D] + b_r)
    z0 = jax.nn.sigmoid(gi[:, _D:2 * _D] + b_z)
    n0 = jnp.tanh(gi[:, 2 * _D:3 * _D] + r0 * b_n)
    h_root = (1.0 - z0) * n0
    isroot = deg_ref[...] == 0
    h0 = jnp.where(isroot, h_root, 0.0)
    h0_ref[...] = h0
    q0_ref[...] = jnp.dot(h0, w2_ref[...], preferred_element_type=jnp.float32)


def _prep(h_prev, wih_t, bih, bhh, w2, deg2d):
    return pl.pallas_call(
        _prep_body,
        grid=(_GB,),
        in_specs=[
            pl.BlockSpec((_BR, _D), lambda i: (i, 0)),
            pl.BlockSpec((_D, 3 * _D), lambda i: (0, 0)),
            pl.BlockSpec((1, 3 * _D), lambda i: (0, 0)),
            pl.BlockSpec((1, 3 * _D), lambda i: (0, 0)),
            pl.BlockSpec((_D, 1), lambda i: (0, 0)),
            pl.BlockSpec((_BR, 1), lambda i: (i, 0)),
        ],
        out_specs=[
            pl.BlockSpec((_BR, 3 * _D), lambda i: (i, 0)),
            pl.BlockSpec((_BR, _D), lambda i: (i, 0)),
            pl.BlockSpec((_BR, 1), lambda i: (i, 0)),
        ],
        out_shape=[
            jax.ShapeDtypeStruct((_NP, 3 * _D), jnp.float32),
            jax.ShapeDtypeStruct((_NP, _D), jnp.float32),
            jax.ShapeDtypeStruct((_NP, 1), jnp.float32),
        ],
    )(h_prev, wih_t, bih, bhh, w2, deg2d)


def _gru_body(gi_ref, h_ref, hp_ref, mr_ref, ss_ref, wt_ref, bhh_ref, w2_ref,
              hn_ref, qn_ref):
    ss = ss_ref[...]
    nm = ss > 0.0
    recip = jnp.where(nm, 1.0 / jnp.where(nm, ss, 1.0), 0.0)
    msg = mr_ref[...] * recip
    gh = jnp.dot(msg, wt_ref[...],
                 preferred_element_type=jnp.float32) + bhh_ref[...]
    gi = gi_ref[...]
    r = jax.nn.sigmoid(gi[:, 0:_D] + gh[:, 0:_D])
    z = jax.nn.sigmoid(gi[:, _D:2 * _D] + gh[:, _D:2 * _D])
    n = jnp.tanh(gi[:, 2 * _D:3 * _D] + r * gh[:, 2 * _D:3 * _D])
    h_upd = (1.0 - z) * n + z * msg
    hn = jnp.where(nm, h_upd, h_ref[...])
    hn_ref[...] = hn
    qn_ref[...] = jnp.dot(hn, w2_ref[...], preferred_element_type=jnp.float32)


def _gru(gi, h, h_prev, msg_raw, ssum2d, whh_t, bhh, w2):
    return pl.pallas_call(
        _gru_body,
        grid=(_GB,),
        in_specs=[
            pl.BlockSpec((_BR, 3 * _D), lambda i: (i, 0)),
            pl.BlockSpec((_BR, _D), lambda i: (i, 0)),
            pl.BlockSpec((_BR, _D), lambda i: (i, 0)),
            pl.BlockSpec((_BR, _D), lambda i: (i, 0)),
            pl.BlockSpec((_BR, 1), lambda i: (i, 0)),
            pl.BlockSpec((_D, 3 * _D), lambda i: (0, 0)),
            pl.BlockSpec((1, 3 * _D), lambda i: (0, 0)),
            pl.BlockSpec((_D, 1), lambda i: (0, 0)),
        ],
        out_specs=[
            pl.BlockSpec((_BR, _D), lambda i: (i, 0)),
            pl.BlockSpec((_BR, 1), lambda i: (i, 0)),
        ],
        out_shape=[
            jax.ShapeDtypeStruct((_NP, _D), jnp.float32),
            jax.ShapeDtypeStruct((_NP, 1), jnp.float32),
        ],
    )(gi, h, h_prev, msg_raw, ssum2d, whh_t, bhh, w2)


def _pool_body(H_ref, b_ref, lc0_ref, lc1_ref, out_ref):
    @pl.when(pl.program_id(0) == 0)
    def _init():
        out_ref[...] = jnp.full((_NGRAPHS, 2 * _D), -jnp.inf, jnp.float32)

    isleaf = (lc0_ref[...] + lc1_ref[...]) == 0.0
    rows = jnp.where(isleaf, H_ref[...], -1e30)
    b = b_ref[...]
    for g in range(_NGRAPHS):
        m = b == g
        colmax = jnp.max(jnp.where(m, rows, -jnp.inf), axis=0)
        out_ref[g, :] = jnp.maximum(out_ref[g, :], colmax)


def _pool(H, batch2d, lc0, lc1):
    return pl.pallas_call(
        _pool_body,
        grid=(_GB,),
        in_specs=[
            pl.BlockSpec((_BR, 2 * _D), lambda i: (i, 0)),
            pl.BlockSpec((_BR, 1), lambda i: (i, 0)),
            pl.BlockSpec((_BR, 1), lambda i: (i, 0)),
            pl.BlockSpec((_BR, 1), lambda i: (i, 0)),
        ],
        out_specs=pl.BlockSpec((_NGRAPHS, 2 * _D), lambda i: (0, 0)),
        out_shape=jax.ShapeDtypeStruct((_NGRAPHS, 2 * _D), jnp.float32),
    )(H, batch2d, lc0, lc1)


def _iota16():
    return lax.broadcasted_iota(jnp.int32, (16,), 0)


def _sget(ref, i, nchunks):
    acc = jnp.int32(0)
    for c in range(nchunks):
        v = ref[pl.ds(c * 16, 16)]
        lane = _iota16() + c * 16
        acc = jnp.maximum(acc, jnp.max(jnp.where(lane == i, v, 0)))
    return acc


def _make_level_sc():
    mesh = plsc.VectorSubcoreMesh(core_axis_name="c", subcore_axis_name="s")

    @functools.partial(
        pl.kernel,
        out_type=[
            jax.ShapeDtypeStruct((_NP, _D), jnp.float32),
            jax.ShapeDtypeStruct((_NP,), jnp.float32),
        ],
        mesh=mesh,
        compiler_params=pltpu.CompilerParams(needs_layout_passes=False),
        scratch_types=[
            pltpu.VMEM((_NP,), jnp.float32),
            pltpu.VMEM((_NLOC, _D), jnp.float32),
            pltpu.VMEM((_NLOC,), jnp.float32),
            pltpu.VMEM((2 * _EK,), jnp.int32),
            pltpu.VMEM((2 * _EK,), jnp.int32),
            pltpu.VMEM((2 * _EK,), jnp.float32),
            pltpu.VMEM((2 * _EK + 16,), jnp.float32),
            pltpu.VMEM((2 * _EK, _D), jnp.float32),
            pltpu.VMEM((48,), jnp.int32),
            pltpu.SemaphoreType.DMA,
            pltpu.SemaphoreType.DMA,
            pltpu.SemaphoreType.DMA,
            pltpu.SemaphoreType.DMA,
        ],
    )
    def level_sc(q_hbm, h_hbm, src_hbm, dst_hbm, g_hbm, eb_hbm, z2_hbm,
                 z1_hbm, msg_hbm, ssum_hbm,
                 q_loc, msg_loc, ssum_loc, src_c, dst_c, g_c, ed_c,
                 rows, ebw, lsem0, lsem1, gsem0, gsem1):
        wid = lax.axis_index("c") * 16 + lax.axis_index("s")
        node_base = pl.multiple_of(wid * _NLOC, _NLOC)
        pltpu.sync_copy(eb_hbm, ebw)
        pltpu.sync_copy(q_hbm, q_loc)
        pltpu.sync_copy(z2_hbm, msg_loc)
        pltpu.sync_copy(z1_hbm, ssum_loc)
        ebpair = ebw[pl.ds(wid, 16)]
        estart = ebpair[0]
        eend = ebpair[1]
        base0 = jnp.bitwise_and(estart, jnp.int32(-8))
        nch = (eend - base0 + _EK - 1) // _EK
        lsems = (lsem0, lsem1)
        gsems = (gsem0, gsem1)

        def issue_linear(ch, p):
            base = pl.multiple_of(base0 + ch * _EK, 8)
            off = p * _EK
            pltpu.async_copy(src_hbm.at[pl.ds(base, _EK)],
                             src_c.at[pl.ds(off, _EK)], lsems[p])
            pltpu.async_copy(dst_hbm.at[pl.ds(base, _EK)],
                             dst_c.at[pl.ds(off, _EK)], lsems[p])
            pltpu.async_copy(g_hbm.at[pl.ds(base, _EK)],
                             g_c.at[pl.ds(off, _EK)], lsems[p])

        def wait_linear(p):
            base = pl.multiple_of(base0, 8)
            off = p * _EK
            pltpu.make_async_copy(src_hbm.at[pl.ds(base, _EK)],
                                  src_c.at[pl.ds(off, _EK)], lsems[p]).wait()
            pltpu.make_async_copy(dst_hbm.at[pl.ds(base, _EK)],
                                  dst_c.at[pl.ds(off, _EK)], lsems[p]).wait()
            pltpu.make_async_copy(g_hbm.at[pl.ds(base, _EK)],
                                  g_c.at[pl.ds(off, _EK)], lsems[p]).wait()

        def issue_gather(p):
            off = p * _EK
            pltpu.async_copy(h_hbm.at[src_c.at[pl.ds(off, _EK)]],
                             rows.at[pl.ds(off, _EK)], gsems[p])

        def wait_gather(p):
            off = p * _EK
            pltpu.make_async_copy(h_hbm.at[src_c.at[pl.ds(off, _EK)]],
                                  rows.at[pl.ds(off, _EK)], gsems[p]).wait()

        def load_linear(ch, p):
            base = pl.multiple_of(base0 + ch * _EK, 8)
            off = p * _EK
            pltpu.sync_copy(src_hbm.at[pl.ds(base, _EK)],
                            src_c.at[pl.ds(off, _EK)])
            pltpu.sync_copy(dst_hbm.at[pl.ds(base, _EK)],
                            dst_c.at[pl.ds(off, _EK)])
            pltpu.sync_copy(g_hbm.at[pl.ds(base, _EK)],
                            g_c.at[pl.ds(off, _EK)])

        load_linear(0, 0)
        issue_gather(0)

        def half_chunk(ch, p, carry):
            base = pl.multiple_of(base0 + ch * _EK, 8)
            off = p * _EK
            load_linear(ch + 1, 1 - p)
            issue_gather(1 - p)
            wait_gather(p)
            for j in range(_EK // 16):
                sl = pl.ds(off + j * 16, 16)
                srcv = src_c[sl]
                qs = plsc.load_gather(q_loc, [srcv])
                eidx = base + j * 16 + _iota16()
                ok = (eidx >= estart) & (eidx < eend)
                ev = jnp.where(ok, g_c[sl] * jnp.exp(qs), 0.0)
                dl = jnp.clip(dst_c[sl] - node_base, 0, _NLOC - 1)
                plsc.addupdate_scatter(ssum_loc, [dl], ev)
                pos = 2 * (j * 16) + 2 * _iota16()
                plsc.store_scatter(ed_c, [pos], ev)
                plsc.store_scatter(ed_c, [pos + 1], dl.astype(jnp.float32))

            def edge_body(i, c2):
                cur_, b0, b1, b2, b3, b4, b5, b6, b7 = c2
                ed = ed_c[pl.ds(2 * i, 16)]
                ev_s = ed[0]
                dl_s = ed[1]

                def flush(cc):
                    cu = cc[0]
                    ci = cu.astype(jnp.int32)
                    msg_loc[ci, pl.ds(0, 16)] = cc[1]
                    msg_loc[ci, pl.ds(16, 16)] = cc[2]
                    msg_loc[ci, pl.ds(32, 16)] = cc[3]
                    msg_loc[ci, pl.ds(48, 16)] = cc[4]
                    msg_loc[ci, pl.ds(64, 16)] = cc[5]
                    msg_loc[ci, pl.ds(80, 16)] = cc[6]
                    msg_loc[ci, pl.ds(96, 16)] = cc[7]
                    msg_loc[ci, pl.ds(112, 16)] = cc[8]
                    z = jnp.zeros((16,), jnp.float32)
                    return (dl_s, z, z, z, z, z, z, z, z)

                cur_, b0, b1, b2, b3, b4, b5, b6, b7 = lax.cond(
                    dl_s != cur_, flush, lambda cc: cc,
                    (cur_, b0, b1, b2, b3, b4, b5, b6, b7))
                evb = jnp.full((16,), ev_s)
                ri = off + i
                b0 = b0 + evb * rows[ri, pl.ds(0, 16)]
                b1 = b1 + evb * rows[ri, pl.ds(16, 16)]
                b2 = b2 + evb * rows[ri, pl.ds(32, 16)]
                b3 = b3 + evb * rows[ri, pl.ds(48, 16)]
                b4 = b4 + evb * rows[ri, pl.ds(64, 16)]
                b5 = b5 + evb * rows[ri, pl.ds(80, 16)]
                b6 = b6 + evb * rows[ri, pl.ds(96, 16)]
                b7 = b7 + evb * rows[ri, pl.ds(112, 16)]
                return (cur_, b0, b1, b2, b3, b4, b5, b6, b7)

            carry = lax.fori_loop(0, _EK, edge_body, carry)
            return carry

        def pair_body(cp, carry):
            carry = half_chunk(2 * cp, 0, carry)
            carry = half_chunk(2 * cp + 1, 1, carry)
            return carry

        npairs = (nch + 1) // 2
        z16 = jnp.zeros((16,), jnp.float32)
        fin = lax.fori_loop(0, npairs, pair_body,
                            (jnp.float32(0.0), z16, z16, z16, z16, z16, z16,
                             z16, z16))
        ci = fin[0].astype(jnp.int32)
        msg_loc[ci, pl.ds(0, 16)] = fin[1]
        msg_loc[ci, pl.ds(16, 16)] = fin[2]
        msg_loc[ci, pl.ds(32, 16)] = fin[3]
        msg_loc[ci, pl.ds(48, 16)] = fin[4]
        msg_loc[ci, pl.ds(64, 16)] = fin[5]
        msg_loc[ci, pl.ds(80, 16)] = fin[6]
        msg_loc[ci, pl.ds(96, 16)] = fin[7]
        msg_loc[ci, pl.ds(112, 16)] = fin[8]
        wait_gather(0)
        pltpu.sync_copy(msg_loc, msg_hbm.at[pl.ds(node_base, _NLOC)])
        pltpu.sync_copy(ssum_loc, ssum_hbm.at[pl.ds(node_base, _NLOC)])

    return level_sc


@functools.lru_cache(maxsize=None)
def _level_sc_cached():
    return _make_level_sc()


def kernel(x, node_depth, edge_index, edge_attr, edge_masks, batch,
           type_emb, attr_emb, depth_emb, edge_W, edge_b, attn_W, attn_b,
           Wih, Whh, bih, bhh):
    src, dst = edge_index[0], edge_index[1]

    order = jnp.argsort(dst)
    src_s = src[order].astype(jnp.int32)
    dst_s = dst[order].astype(jnp.int32)
    ea_s = edge_attr[order]
    mf_s = edge_masks[:, order].astype(jnp.float32)
    row_ptr = jnp.searchsorted(dst_s, jnp.arange(_NP + 1, dtype=jnp.int32),
                               ).astype(jnp.int32)
    deg2d = (row_ptr[1:] - row_ptr[:-1]).reshape(_NP, 1)
    eb = row_ptr[jnp.arange(_NW + 1) * _NLOC]
    eb = jnp.pad(eb, (0, 15)).astype(jnp.int32)

    padE = _EP - _E
    src_p = jnp.pad(src_s, (0, padE))
    dst_p = jnp.pad(dst_s, (0, padE), constant_values=_NP - 1)
    ea0 = jnp.pad(ea_s[:, 0], (0, padE)).reshape(_ER, 128)
    ea1 = jnp.pad(ea_s[:, 1], (0, padE)).reshape(_ER, 128)
    mfp = jnp.pad(mf_s, ((0, 0), (0, padE))).reshape(8, _ER, 128)

    depth = jnp.minimum(node_depth, _MAXD)
    h_prev = type_emb[x[:, 0]] + attr_emb[x[:, 1]] + depth_emb[depth]
    h_prev = jnp.pad(h_prev, ((0, _NP - _N), (0, 0)))
    lcnt = jnp.zeros((_NP,), jnp.float32).at[src].add(1.0)
    lc0 = lcnt.reshape(_NP, 1)
    lc1 = jnp.zeros((_NP, 1), jnp.float32)

    w2s = attn_W[:, _D:, 0]
    ew2 = jnp.einsum('lcd,ld->lc', edge_W, w2s)
    g_all = _coef(ew2, ea0, ea1, mfp)
    g2 = g_all.reshape(2, 8, _EP)
    z2 = jnp.zeros((_NLOC, _D), jnp.float32)
    z1 = jnp.zeros((_NLOC,), jnp.float32)
    level_sc = _level_sc_cached()

    batch2d = jnp.pad(batch, (0, _NP - _N),
                      constant_values=_NGRAPHS).astype(jnp.int32).reshape(_NP, 1)

    h_layers = []
    for l in range(_NLAYERS):
        w2 = attn_W[l, _D:, :]
        wih_t = Wih[l].T
        whh_t = Whh[l].T
        bih2 = bih[l].reshape(1, 3 * _D)
        bhh2 = bhh[l].reshape(1, 3 * _D)
        gi, h, q = _prep(h_prev, wih_t, bih2, bhh2, w2, deg2d)
        for lev in range(_NLEVELS):
            msg_raw, ssum = level_sc(q.reshape(_NP), h, src_p, dst_p,
                                     g2[l, lev], eb, z2, z1)
            h, q = _gru(gi, h, h_prev, msg_raw, ssum.reshape(_NP, 1),
                        whh_t, bhh2, w2)
        h_layers.append(h)
        h_prev = h

    H = jnp.concatenate(h_layers, axis=1)
    return _pool(H, batch2d, lc0, lc1)


# speedup vs baseline: 8.6104x; 1.0745x over previous
import functools
import jax
import jax.numpy as jnp
from jax import lax
from jax.experimental import pallas as pl
from jax.experimental.pallas import tpu as pltpu
from jax.experimental.pallas import tpu_sc as plsc

_N = 10000
_E = 320000
_D = 128
_NLAYERS = 2
_NLEVELS = 8
_MAXD = 20
_NGRAPHS = 16

_NW = 32
_NLOC = 320
_NP = _NW * _NLOC
_EK = 128
_EP = _E + 4 * _EK
_ER = _EP // 128

_BR = 2048
_GB = _NP // _BR


def _coef_body(sc_ref, ea0_ref, ea1_ref, out_ref):
    l = pl.program_id(0)
    s0 = sc_ref[l, 0]
    s1 = sc_ref[l, 1]
    out_ref[0] = jnp.exp(ea0_ref[...] * s0 + ea1_ref[...] * s1)


def _coef(ew2, ea0, ea1):
    return pl.pallas_call(
        _coef_body,
        grid=(2,),
        in_specs=[
            pl.BlockSpec(memory_space=pltpu.SMEM),
            pl.BlockSpec((_ER, 128), lambda l: (0, 0)),
            pl.BlockSpec((_ER, 128), lambda l: (0, 0)),
        ],
        out_specs=pl.BlockSpec((1, _ER, 128), lambda l: (l, 0, 0)),
        out_shape=jax.ShapeDtypeStruct((2, _ER, 128), jnp.float32),
    )(ew2, ea0, ea1)


def _prep_body(hp_ref, wt_ref, bih_ref, bhh_ref, w2_ref, deg_ref,
               gi_ref, h0_ref, q0_ref):
    gi = jnp.dot(hp_ref[...], wt_ref[...],
                 preferred_element_type=jnp.float32) + bih_ref[...]
    gi_ref[...] = gi
    b_r = bhh_ref[0, 0:_D]
    b_z = bhh_ref[0, _D:2 * _D]
    b_n = bhh_ref[0, 2 * _D:3 * _D]
    r0 = jax.nn.sigmoid(gi[:, 0:_D] + b_r)
    z0 = jax.nn.sigmoid(gi[:, _D:2 * _D] + b_z)
    n0 = jnp.tanh(gi[:, 2 * _D:3 * _D] + r0 * b_n)
    h_root = (1.0 - z0) * n0
    isroot = deg_ref[...] == 0
    h0 = jnp.where(isroot, h_root, 0.0)
    h0_ref[...] = h0
    q0_ref[...] = jnp.dot(h0, w2_ref[...], preferred_element_type=jnp.float32)


def _prep(h_prev, wih_t, bih, bhh, w2, deg2d):
    return pl.pallas_call(
        _prep_body,
        grid=(_GB,),
        in_specs=[
            pl.BlockSpec((_BR, _D), lambda i: (i, 0)),
            pl.BlockSpec((_D, 3 * _D), lambda i: (0, 0)),
            pl.BlockSpec((1, 3 * _D), lambda i: (0, 0)),
            pl.BlockSpec((1, 3 * _D), lambda i: (0, 0)),
            pl.BlockSpec((_D, 1), lambda i: (0, 0)),
            pl.BlockSpec((_BR, 1), lambda i: (i, 0)),
        ],
        out_specs=[
            pl.BlockSpec((_BR, 3 * _D), lambda i: (i, 0)),
            pl.BlockSpec((_BR, _D), lambda i: (i, 0)),
            pl.BlockSpec((_BR, 1), lambda i: (i, 0)),
        ],
        out_shape=[
            jax.ShapeDtypeStruct((_NP, 3 * _D), jnp.float32),
            jax.ShapeDtypeStruct((_NP, _D), jnp.float32),
            jax.ShapeDtypeStruct((_NP, 1), jnp.float32),
        ],
    )(h_prev, wih_t, bih, bhh, w2, deg2d)


def _gru_body(gi_ref, h_ref, hp_ref, mr_ref, ss_ref, wt_ref, bhh_ref, w2_ref,
              hn_ref, qn_ref):
    ss = ss_ref[...]
    nm = ss > 0.0
    recip = jnp.where(nm, 1.0 / jnp.where(nm, ss, 1.0), 0.0)
    msg = mr_ref[...] * recip
    gh = jnp.dot(msg, wt_ref[...],
                 preferred_element_type=jnp.float32) + bhh_ref[...]
    gi = gi_ref[...]
    r = jax.nn.sigmoid(gi[:, 0:_D] + gh[:, 0:_D])
    z = jax.nn.sigmoid(gi[:, _D:2 * _D] + gh[:, _D:2 * _D])
    n = jnp.tanh(gi[:, 2 * _D:3 * _D] + r * gh[:, 2 * _D:3 * _D])
    h_upd = (1.0 - z) * n + z * msg
    hn = jnp.where(nm, h_upd, h_ref[...])
    hn_ref[...] = hn
    qn_ref[...] = jnp.dot(hn, w2_ref[...], preferred_element_type=jnp.float32)


def _gru(gi, h, h_prev, msg_raw, ssum2d, whh_t, bhh, w2):
    return pl.pallas_call(
        _gru_body,
        grid=(_GB,),
        in_specs=[
            pl.BlockSpec((_BR, 3 * _D), lambda i: (i, 0)),
            pl.BlockSpec((_BR, _D), lambda i: (i, 0)),
            pl.BlockSpec((_BR, _D), lambda i: (i, 0)),
            pl.BlockSpec((_BR, _D), lambda i: (i, 0)),
            pl.BlockSpec((_BR, 1), lambda i: (i, 0)),
            pl.BlockSpec((_D, 3 * _D), lambda i: (0, 0)),
            pl.BlockSpec((1, 3 * _D), lambda i: (0, 0)),
            pl.BlockSpec((_D, 1), lambda i: (0, 0)),
        ],
        out_specs=[
            pl.BlockSpec((_BR, _D), lambda i: (i, 0)),
            pl.BlockSpec((_BR, 1), lambda i: (i, 0)),
        ],
        out_shape=[
            jax.ShapeDtypeStruct((_NP, _D), jnp.float32),
            jax.ShapeDtypeStruct((_NP, 1), jnp.float32),
        ],
    )(gi, h, h_prev, msg_raw, ssum2d, whh_t, bhh, w2)


def _pool_body(H_ref, b_ref, lc0_ref, lc1_ref, out_ref):
    @pl.when(pl.program_id(0) == 0)
    def _init():
        out_ref[...] = jnp.full((_NGRAPHS, 2 * _D), -jnp.inf, jnp.float32)

    isleaf = (lc0_ref[...] + lc1_ref[...]) == 0.0
    rows = jnp.where(isleaf, H_ref[...], -1e30)
    b = b_ref[...]
    for g in range(_NGRAPHS):
        m = b == g
        colmax = jnp.max(jnp.where(m, rows, -jnp.inf), axis=0)
        out_ref[g, :] = jnp.maximum(out_ref[g, :], colmax)


def _pool(H, batch2d, lc0, lc1):
    return pl.pallas_call(
        _pool_body,
        grid=(_GB,),
        in_specs=[
            pl.BlockSpec((_BR, 2 * _D), lambda i: (i, 0)),
            pl.BlockSpec((_BR, 1), lambda i: (i, 0)),
            pl.BlockSpec((_BR, 1), lambda i: (i, 0)),
            pl.BlockSpec((_BR, 1), lambda i: (i, 0)),
        ],
        out_specs=pl.BlockSpec((_NGRAPHS, 2 * _D), lambda i: (0, 0)),
        out_shape=jax.ShapeDtypeStruct((_NGRAPHS, 2 * _D), jnp.float32),
    )(H, batch2d, lc0, lc1)


def _iota16():
    return lax.broadcasted_iota(jnp.int32, (16,), 0)


def _sget(ref, i, nchunks):
    acc = jnp.int32(0)
    for c in range(nchunks):
        v = ref[pl.ds(c * 16, 16)]
        lane = _iota16() + c * 16
        acc = jnp.maximum(acc, jnp.max(jnp.where(lane == i, v, 0)))
    return acc


def _make_level_sc():
    mesh = plsc.VectorSubcoreMesh(core_axis_name="c", subcore_axis_name="s")

    @functools.partial(
        pl.kernel,
        out_type=[
            jax.ShapeDtypeStruct((_NP, _D), jnp.float32),
            jax.ShapeDtypeStruct((_NP,), jnp.float32),
        ],
        mesh=mesh,
        compiler_params=pltpu.CompilerParams(needs_layout_passes=False),
        scratch_types=[
            pltpu.VMEM((_NP,), jnp.float32),
            pltpu.VMEM((_NLOC, _D), jnp.float32),
            pltpu.VMEM((_NLOC,), jnp.float32),
            pltpu.VMEM((2 * _EK,), jnp.int32),
            pltpu.VMEM((2 * _EK,), jnp.int32),
            pltpu.VMEM((2 * _EK,), jnp.float32),
            pltpu.VMEM((2 * _EK,), jnp.int32),
            pltpu.VMEM((2 * _EK + 16,), jnp.float32),
            pltpu.VMEM((2 * _EK, _D), jnp.float32),
            pltpu.VMEM((48,), jnp.int32),
            pltpu.SemaphoreType.DMA,
            pltpu.SemaphoreType.DMA,
        ],
    )
    def level_sc(q_hbm, h_hbm, src_hbm, dst_hbm, g_hbm, mp_hbm, eb_hbm,
                 z2_hbm, z1_hbm, msg_hbm, ssum_hbm,
                 q_loc, msg_loc, ssum_loc, src_c, dst_c, g_c, mp_c, ed_c,
                 rows, ebw, gsem0, gsem1):
        wid = lax.axis_index("c") * 16 + lax.axis_index("s")
        node_base = pl.multiple_of(wid * _NLOC, _NLOC)
        pltpu.sync_copy(eb_hbm, ebw)
        pltpu.sync_copy(q_hbm, q_loc)
        pltpu.sync_copy(z2_hbm, msg_loc)
        pltpu.sync_copy(z1_hbm, ssum_loc)
        ebpair = ebw[pl.ds(wid, 16)]
        estart = ebpair[0]
        eend = ebpair[1]
        lev = ebw[pl.ds(32, 16)][8]
        base0 = jnp.bitwise_and(estart, jnp.int32(-8))
        nch = (eend - base0 + _EK - 1) // _EK
        gsems = (gsem0, gsem1)

        def issue_gather(p):
            off = p * _EK
            pltpu.async_copy(h_hbm.at[src_c.at[pl.ds(off, _EK)]],
                             rows.at[pl.ds(off, _EK)], gsems[p])

        def wait_gather(p):
            off = p * _EK
            pltpu.make_async_copy(h_hbm.at[src_c.at[pl.ds(off, _EK)]],
                                  rows.at[pl.ds(off, _EK)], gsems[p]).wait()

        def load_linear(ch, p):
            base = pl.multiple_of(base0 + ch * _EK, 8)
            off = p * _EK
            pltpu.sync_copy(src_hbm.at[pl.ds(base, _EK)],
                            src_c.at[pl.ds(off, _EK)])
            pltpu.sync_copy(dst_hbm.at[pl.ds(base, _EK)],
                            dst_c.at[pl.ds(off, _EK)])
            pltpu.sync_copy(g_hbm.at[pl.ds(base, _EK)],
                            g_c.at[pl.ds(off, _EK)])
            pltpu.sync_copy(mp_hbm.at[pl.ds(base, _EK)],
                            mp_c.at[pl.ds(off, _EK)])

        load_linear(0, 0)
        issue_gather(0)

        def half_chunk(ch, p, carry):
            base = pl.multiple_of(base0 + ch * _EK, 8)
            off = p * _EK
            load_linear(ch + 1, 1 - p)
            issue_gather(1 - p)
            wait_gather(p)
            for j in range(_EK // 16):
                sl = pl.ds(off + j * 16, 16)
                srcv = src_c[sl]
                qs = plsc.load_gather(q_loc, [srcv])
                eidx = base + j * 16 + _iota16()
                mbit = jnp.bitwise_and(
                    jnp.right_shift(mp_c[sl], jnp.full((16,), lev)), 1)
                ok = (eidx >= estart) & (eidx < eend) & (mbit > 0)
                ev = jnp.where(ok, g_c[sl] * jnp.exp(qs), 0.0)
                dl = jnp.clip(dst_c[sl] - node_base, 0, _NLOC - 1)
                plsc.addupdate_scatter(ssum_loc, [dl], ev)
                pos = 2 * (j * 16) + 2 * _iota16()
                plsc.store_scatter(ed_c, [pos], ev)
                plsc.store_scatter(ed_c, [pos + 1], dl.astype(jnp.float32))

            def edge_body(i, c2):
                cur_, b0, b1, b2, b3, b4, b5, b6, b7 = c2
                ed = ed_c[pl.ds(2 * i, 16)]
                ev_s = ed[0]
                dl_s = ed[1]

                def flush(cc):
                    cu = cc[0]
                    ci = cu.astype(jnp.int32)
                    msg_loc[ci, pl.ds(0, 16)] = cc[1]
                    msg_loc[ci, pl.ds(16, 16)] = cc[2]
                    msg_loc[ci, pl.ds(32, 16)] = cc[3]
                    msg_loc[ci, pl.ds(48, 16)] = cc[4]
                    msg_loc[ci, pl.ds(64, 16)] = cc[5]
                    msg_loc[ci, pl.ds(80, 16)] = cc[6]
                    msg_loc[ci, pl.ds(96, 16)] = cc[7]
                    msg_loc[ci, pl.ds(112, 16)] = cc[8]
                    z = jnp.zeros((16,), jnp.float32)
                    return (dl_s, z, z, z, z, z, z, z, z)

                cur_, b0, b1, b2, b3, b4, b5, b6, b7 = lax.cond(
                    dl_s != cur_, flush, lambda cc: cc,
                    (cur_, b0, b1, b2, b3, b4, b5, b6, b7))
                evb = jnp.full((16,), ev_s)
                ri = off + i
                b0 = b0 + evb * rows[ri, pl.ds(0, 16)]
                b1 = b1 + evb * rows[ri, pl.ds(16, 16)]
                b2 = b2 + evb * rows[ri, pl.ds(32, 16)]
                b3 = b3 + evb * rows[ri, pl.ds(48, 16)]
                b4 = b4 + evb * rows[ri, pl.ds(64, 16)]
                b5 = b5 + evb * rows[ri, pl.ds(80, 16)]
                b6 = b6 + evb * rows[ri, pl.ds(96, 16)]
                b7 = b7 + evb * rows[ri, pl.ds(112, 16)]
                return (cur_, b0, b1, b2, b3, b4, b5, b6, b7)

            carry = lax.fori_loop(0, _EK, edge_body, carry)
            return carry

        def pair_body(cp, carry):
            carry = half_chunk(2 * cp, 0, carry)
            carry = half_chunk(2 * cp + 1, 1, carry)
            return carry

        npairs = (nch + 1) // 2
        z16 = jnp.zeros((16,), jnp.float32)
        fin = lax.fori_loop(0, npairs, pair_body,
                            (jnp.float32(0.0), z16, z16, z16, z16, z16, z16,
                             z16, z16))
        ci = fin[0].astype(jnp.int32)
        msg_loc[ci, pl.ds(0, 16)] = fin[1]
        msg_loc[ci, pl.ds(16, 16)] = fin[2]
        msg_loc[ci, pl.ds(32, 16)] = fin[3]
        msg_loc[ci, pl.ds(48, 16)] = fin[4]
        msg_loc[ci, pl.ds(64, 16)] = fin[5]
        msg_loc[ci, pl.ds(80, 16)] = fin[6]
        msg_loc[ci, pl.ds(96, 16)] = fin[7]
        msg_loc[ci, pl.ds(112, 16)] = fin[8]
        wait_gather(0)
        pltpu.sync_copy(msg_loc, msg_hbm.at[pl.ds(node_base, _NLOC)])
        pltpu.sync_copy(ssum_loc, ssum_hbm.at[pl.ds(node_base, _NLOC)])

    return level_sc


@functools.lru_cache(maxsize=None)
def _level_sc_cached():
    return _make_level_sc()


def kernel(x, node_depth, edge_index, edge_attr, edge_masks, batch,
           type_emb, attr_emb, depth_emb, edge_W, edge_b, attn_W, attn_b,
           Wih, Whh, bih, bhh):
    src, dst = edge_index[0], edge_index[1]

    order = jnp.argsort(dst)
    src_s = src[order].astype(jnp.int32)
    dst_s = dst[order].astype(jnp.int32)
    ea_s = edge_attr[order]
    mpow = (jnp.int32(1) << jnp.arange(8, dtype=jnp.int32))[:, None]
    mpack = jnp.sum(edge_masks.astype(jnp.int32) * mpow, axis=0,
                    dtype=jnp.int32)
    mp_s = mpack[order]
    row_ptr = jnp.searchsorted(dst_s, jnp.arange(_NP + 1, dtype=jnp.int32),
                               ).astype(jnp.int32)
    deg2d = (row_ptr[1:] - row_ptr[:-1]).reshape(_NP, 1)
    eb = row_ptr[jnp.arange(_NW + 1) * _NLOC]
    eb = jnp.pad(eb, (0, 15)).astype(jnp.int32)
    eb_levs = [eb.at[40].set(lev) for lev in range(_NLEVELS)]

    padE = _EP - _E
    src_p = jnp.pad(src_s, (0, padE))
    dst_p = jnp.pad(dst_s, (0, padE), constant_values=_NP - 1)
    mp_p = jnp.pad(mp_s, (0, padE))
    ea0 = jnp.pad(ea_s[:, 0], (0, padE)).reshape(_ER, 128)
    ea1 = jnp.pad(ea_s[:, 1], (0, padE)).reshape(_ER, 128)

    depth = jnp.minimum(node_depth, _MAXD)
    h_prev = type_emb[x[:, 0]] + attr_emb[x[:, 1]] + depth_emb[depth]
    h_prev = jnp.pad(h_prev, ((0, _NP - _N), (0, 0)))
    lcnt = jnp.zeros((_NP,), jnp.float32).at[src].add(1.0)
    lc0 = lcnt.reshape(_NP, 1)
    lc1 = jnp.zeros((_NP, 1), jnp.float32)

    w2s = attn_W[:, _D:, 0]
    ew2 = jnp.einsum('lcd,ld->lc', edge_W, w2s)
    g_all = _coef(ew2, ea0, ea1)
    g2 = g_all.reshape(2, _EP)
    z2 = jnp.zeros((_NLOC, _D), jnp.float32)
    z1 = jnp.zeros((_NLOC,), jnp.float32)
    level_sc = _level_sc_cached()

    batch2d = jnp.pad(batch, (0, _NP - _N),
                      constant_values=_NGRAPHS).astype(jnp.int32).reshape(_NP, 1)

    h_layers = []
    for l in range(_NLAYERS):
        w2 = attn_W[l, _D:, :]
        wih_t = Wih[l].T
        whh_t = Whh[l].T
        bih2 = bih[l].reshape(1, 3 * _D)
        bhh2 = bhh[l].reshape(1, 3 * _D)
        gi, h, q = _prep(h_prev, wih_t, bih2, bhh2, w2, deg2d)
        for lev in range(_NLEVELS):
            msg_raw, ssum = level_sc(q.reshape(_NP), h, src_p, dst_p,
                                     g2[l], mp_p, eb_levs[lev], z2, z1)
            h, q = _gru(gi, h, h_prev, msg_raw, ssum.reshape(_NP, 1),
                        whh_t, bhh2, w2)
        h_layers.append(h)
        h_prev = h

    H = jnp.concatenate(h_layers, axis=1)
    return _pool(H, batch2d, lc0, lc1)


# speedup vs baseline: 8.6831x; 1.0084x over previous
import functools
import jax
import jax.numpy as jnp
from jax import lax
from jax.experimental import pallas as pl
from jax.experimental.pallas import tpu as pltpu
from jax.experimental.pallas import tpu_sc as plsc

_N = 10000
_E = 320000
_D = 128
_NLAYERS = 2
_NLEVELS = 8
_MAXD = 20
_NGRAPHS = 16

_NW = 32
_NLOC = 320
_NP = _NW * _NLOC
_EK = 128
_EP = _E + 4 * _EK
_ER = _EP // 128

_BR = 2048
_GB = _NP // _BR


def _coef_body(sc_ref, ea0_ref, ea1_ref, out_ref):
    l = pl.program_id(0)
    s0 = sc_ref[l, 0]
    s1 = sc_ref[l, 1]
    out_ref[0] = jnp.exp(ea0_ref[...] * s0 + ea1_ref[...] * s1)


def _coef(ew2, ea0, ea1):
    return pl.pallas_call(
        _coef_body,
        grid=(2,),
        in_specs=[
            pl.BlockSpec(memory_space=pltpu.SMEM),
            pl.BlockSpec((_ER, 128), lambda l: (0, 0)),
            pl.BlockSpec((_ER, 128), lambda l: (0, 0)),
        ],
        out_specs=pl.BlockSpec((1, _ER, 128), lambda l: (l, 0, 0)),
        out_shape=jax.ShapeDtypeStruct((2, _ER, 128), jnp.float32),
    )(ew2, ea0, ea1)


def _prep_body(hp_ref, wt_ref, bih_ref, bhh_ref, w2_ref, deg_ref,
               gi_ref, h0_ref, q0_ref):
    gi = jnp.dot(hp_ref[...], wt_ref[...],
                 preferred_element_type=jnp.float32) + bih_ref[...]
    gi_ref[...] = gi
    b_r = bhh_ref[0, 0:_D]
    b_z = bhh_ref[0, _D:2 * _D]
    b_n = bhh_ref[0, 2 * _D:3 * _D]
    r0 = jax.nn.sigmoid(gi[:, 0:_D] + b_r)
    z0 = jax.nn.sigmoid(gi[:, _D:2 * _D] + b_z)
    n0 = jnp.tanh(gi[:, 2 * _D:3 * _D] + r0 * b_n)
    h_root = (1.0 - z0) * n0
    isroot = deg_ref[...] == 0
    h0 = jnp.where(isroot, h_root, 0.0)
    h0_ref[...] = h0
    q0_ref[...] = jnp.dot(h0, w2_ref[...], preferred_element_type=jnp.float32)


def _prep(h_prev, wih_t, bih, bhh, w2, deg2d):
    return pl.pallas_call(
        _prep_body,
        grid=(_GB,),
        in_specs=[
            pl.BlockSpec((_BR, _D), lambda i: (i, 0)),
            pl.BlockSpec((_D, 3 * _D), lambda i: (0, 0)),
            pl.BlockSpec((1, 3 * _D), lambda i: (0, 0)),
            pl.BlockSpec((1, 3 * _D), lambda i: (0, 0)),
            pl.BlockSpec((_D, 1), lambda i: (0, 0)),
            pl.BlockSpec((_BR, 1), lambda i: (i, 0)),
        ],
        out_specs=[
            pl.BlockSpec((_BR, 3 * _D), lambda i: (i, 0)),
            pl.BlockSpec((_BR, _D), lambda i: (i, 0)),
            pl.BlockSpec((_BR, 1), lambda i: (i, 0)),
        ],
        out_shape=[
            jax.ShapeDtypeStruct((_NP, 3 * _D), jnp.float32),
            jax.ShapeDtypeStruct((_NP, _D), jnp.float32),
            jax.ShapeDtypeStruct((_NP, 1), jnp.float32),
        ],
    )(h_prev, wih_t, bih, bhh, w2, deg2d)


def _gru_body(gi_ref, h_ref, hp_ref, mr_ref, ss_ref, wt_ref, bhh_ref, w2_ref,
              hn_ref, qn_ref):
    ss = ss_ref[...]
    nm = ss > 0.0
    recip = jnp.where(nm, 1.0 / jnp.where(nm, ss, 1.0), 0.0)
    msg = mr_ref[...] * recip
    gh = jnp.dot(msg, wt_ref[...],
                 preferred_element_type=jnp.float32) + bhh_ref[...]
    gi = gi_ref[...]
    r = jax.nn.sigmoid(gi[:, 0:_D] + gh[:, 0:_D])
    z = jax.nn.sigmoid(gi[:, _D:2 * _D] + gh[:, _D:2 * _D])
    n = jnp.tanh(gi[:, 2 * _D:3 * _D] + r * gh[:, 2 * _D:3 * _D])
    h_upd = (1.0 - z) * n + z * msg
    hn = jnp.where(nm, h_upd, h_ref[...])
    hn_ref[...] = hn
    qn_ref[...] = jnp.dot(hn, w2_ref[...], preferred_element_type=jnp.float32)


def _gru(gi, h, h_prev, msg_raw, ssum2d, whh_t, bhh, w2):
    return pl.pallas_call(
        _gru_body,
        grid=(_GB,),
        in_specs=[
            pl.BlockSpec((_BR, 3 * _D), lambda i: (i, 0)),
            pl.BlockSpec((_BR, _D), lambda i: (i, 0)),
            pl.BlockSpec((_BR, _D), lambda i: (i, 0)),
            pl.BlockSpec((_BR, _D), lambda i: (i, 0)),
            pl.BlockSpec((_BR, 1), lambda i: (i, 0)),
            pl.BlockSpec((_D, 3 * _D), lambda i: (0, 0)),
            pl.BlockSpec((1, 3 * _D), lambda i: (0, 0)),
            pl.BlockSpec((_D, 1), lambda i: (0, 0)),
        ],
        out_specs=[
            pl.BlockSpec((_BR, _D), lambda i: (i, 0)),
            pl.BlockSpec((_BR, 1), lambda i: (i, 0)),
        ],
        out_shape=[
            jax.ShapeDtypeStruct((_NP, _D), jnp.float32),
            jax.ShapeDtypeStruct((_NP, 1), jnp.float32),
        ],
    )(gi, h, h_prev, msg_raw, ssum2d, whh_t, bhh, w2)


def _pool_body(H_ref, b_ref, lc0_ref, lc1_ref, out_ref):
    @pl.when(pl.program_id(0) == 0)
    def _init():
        out_ref[...] = jnp.full((_NGRAPHS, 2 * _D), -jnp.inf, jnp.float32)

    isleaf = (lc0_ref[...] + lc1_ref[...]) == 0.0
    rows = jnp.where(isleaf, H_ref[...], -1e30)
    b = b_ref[...]
    for g in range(_NGRAPHS):
        m = b == g
        colmax = jnp.max(jnp.where(m, rows, -jnp.inf), axis=0)
        out_ref[g, :] = jnp.maximum(out_ref[g, :], colmax)


def _pool(H, batch2d, lc0, lc1):
    return pl.pallas_call(
        _pool_body,
        grid=(_GB,),
        in_specs=[
            pl.BlockSpec((_BR, 2 * _D), lambda i: (i, 0)),
            pl.BlockSpec((_BR, 1), lambda i: (i, 0)),
            pl.BlockSpec((_BR, 1), lambda i: (i, 0)),
            pl.BlockSpec((_BR, 1), lambda i: (i, 0)),
        ],
        out_specs=pl.BlockSpec((_NGRAPHS, 2 * _D), lambda i: (0, 0)),
        out_shape=jax.ShapeDtypeStruct((_NGRAPHS, 2 * _D), jnp.float32),
    )(H, batch2d, lc0, lc1)


def _iota16():
    return lax.broadcasted_iota(jnp.int32, (16,), 0)


def _sget(ref, i, nchunks):
    acc = jnp.int32(0)
    for c in range(nchunks):
        v = ref[pl.ds(c * 16, 16)]
        lane = _iota16() + c * 16
        acc = jnp.maximum(acc, jnp.max(jnp.where(lane == i, v, 0)))
    return acc


def _make_level_sc():
    mesh = plsc.VectorSubcoreMesh(core_axis_name="c", subcore_axis_name="s")

    @functools.partial(
        pl.kernel,
        out_type=[
            jax.ShapeDtypeStruct((_NP, _D), jnp.float32),
            jax.ShapeDtypeStruct((_NP,), jnp.float32),
        ],
        mesh=mesh,
        compiler_params=pltpu.CompilerParams(needs_layout_passes=False),
        scratch_types=[
            pltpu.VMEM((_NP,), jnp.float32),
            pltpu.VMEM((_NLOC, _D), jnp.float32),
            pltpu.VMEM((_NLOC,), jnp.float32),
            pltpu.VMEM((2 * _EK,), jnp.int32),
            pltpu.VMEM((2 * _EK,), jnp.int32),
            pltpu.VMEM((2 * _EK,), jnp.float32),
            pltpu.VMEM((2 * _EK,), jnp.int32),
            pltpu.VMEM((2 * _EK + 16,), jnp.float32),
            pltpu.VMEM((2 * _EK, _D), jnp.float32),
            pltpu.VMEM((48,), jnp.int32),
            pltpu.SemaphoreType.DMA,
            pltpu.SemaphoreType.DMA,
        ],
    )
    def level_sc(q_hbm, h_hbm, src_hbm, dst_hbm, g_hbm, mp_hbm, eb_hbm,
                 z2_hbm, z1_hbm, msg_hbm, ssum_hbm,
                 q_loc, msg_loc, ssum_loc, src_c, dst_c, g_c, mp_c, ed_c,
                 rows, ebw, gsem0, gsem1):
        wid = lax.axis_index("c") * 16 + lax.axis_index("s")
        node_base = pl.multiple_of(wid * _NLOC, _NLOC)
        pltpu.sync_copy(eb_hbm, ebw)
        pltpu.sync_copy(q_hbm, q_loc)
        pltpu.sync_copy(z2_hbm, msg_loc)
        pltpu.sync_copy(z1_hbm, ssum_loc)
        ebpair = ebw[pl.ds(wid, 16)]
        estart = ebpair[0]
        eend = ebpair[1]
        lev = ebw[pl.ds(32, 16)][8]
        base0 = jnp.bitwise_and(estart, jnp.int32(-8))
        nch = (eend - base0 + _EK - 1) // _EK
        gsems = (gsem0, gsem1)

        def issue_gather(p):
            off = p * _EK
            pltpu.async_copy(h_hbm.at[src_c.at[pl.ds(off, _EK)]],
                             rows.at[pl.ds(off, _EK)], gsems[p])

        def wait_gather(p):
            off = p * _EK
            pltpu.make_async_copy(h_hbm.at[src_c.at[pl.ds(off, _EK)]],
                                  rows.at[pl.ds(off, _EK)], gsems[p]).wait()

        def load_linear(ch, p):
            base = pl.multiple_of(base0 + ch * _EK, 8)
            off = p * _EK
            pltpu.sync_copy(src_hbm.at[pl.ds(base, _EK)],
                            src_c.at[pl.ds(off, _EK)])
            pltpu.sync_copy(dst_hbm.at[pl.ds(base, _EK)],
                            dst_c.at[pl.ds(off, _EK)])
            pltpu.sync_copy(g_hbm.at[pl.ds(base, _EK)],
                            g_c.at[pl.ds(off, _EK)])
            pltpu.sync_copy(mp_hbm.at[pl.ds(base, _EK)],
                            mp_c.at[pl.ds(off, _EK)])

        load_linear(0, 0)
        issue_gather(0)

        def half_chunk(ch, p, carry):
            base = pl.multiple_of(base0 + ch * _EK, 8)
            off = p * _EK
            load_linear(ch + 1, 1 - p)
            issue_gather(1 - p)
            wait_gather(p)
            for j in range(_EK // 16):
                sl = pl.ds(off + j * 16, 16)
                srcv = src_c[sl]
                qs = plsc.load_gather(q_loc, [srcv])
                eidx = base + j * 16 + _iota16()
                mbit = jnp.bitwise_and(
                    jnp.right_shift(mp_c[sl], jnp.full((16,), lev)), 1)
                ok = (eidx >= estart) & (eidx < eend) & (mbit > 0)
                ev = jnp.where(ok, g_c[sl] * jnp.exp(qs), 0.0)
                dl = jnp.clip(dst_c[sl] - node_base, 0, _NLOC - 1)
                plsc.addupdate_scatter(ssum_loc, [dl], ev)
                pos = 2 * (j * 16) + 2 * _iota16()
                plsc.store_scatter(ed_c, [pos], ev)
                plsc.store_scatter(ed_c, [pos + 1], dl.astype(jnp.float32))

            def edge_body(i, c2):
                cur_, b0, b1, b2, b3, b4, b5, b6, b7 = c2
                ed = ed_c[pl.ds(2 * i, 16)]
                ev_s = ed[0]
                dl_s = ed[1]

                def flush(cc):
                    cu = cc[0]
                    ci = cu.astype(jnp.int32)
                    msg_loc[ci, pl.ds(0, 16)] = cc[1]
                    msg_loc[ci, pl.ds(16, 16)] = cc[2]
                    msg_loc[ci, pl.ds(32, 16)] = cc[3]
                    msg_loc[ci, pl.ds(48, 16)] = cc[4]
                    msg_loc[ci, pl.ds(64, 16)] = cc[5]
                    msg_loc[ci, pl.ds(80, 16)] = cc[6]
                    msg_loc[ci, pl.ds(96, 16)] = cc[7]
                    msg_loc[ci, pl.ds(112, 16)] = cc[8]
                    z = jnp.zeros((16,), jnp.float32)
                    return (dl_s, z, z, z, z, z, z, z, z)

                cur_, b0, b1, b2, b3, b4, b5, b6, b7 = lax.cond(
                    dl_s != cur_, flush, lambda cc: cc,
                    (cur_, b0, b1, b2, b3, b4, b5, b6, b7))
                evb = jnp.full((16,), ev_s)
                ri = off + i
                b0 = b0 + evb * rows[ri, pl.ds(0, 16)]
                b1 = b1 + evb * rows[ri, pl.ds(16, 16)]
                b2 = b2 + evb * rows[ri, pl.ds(32, 16)]
                b3 = b3 + evb * rows[ri, pl.ds(48, 16)]
                b4 = b4 + evb * rows[ri, pl.ds(64, 16)]
                b5 = b5 + evb * rows[ri, pl.ds(80, 16)]
                b6 = b6 + evb * rows[ri, pl.ds(96, 16)]
                b7 = b7 + evb * rows[ri, pl.ds(112, 16)]
                return (cur_, b0, b1, b2, b3, b4, b5, b6, b7)

            carry = lax.fori_loop(0, _EK, edge_body, carry)
            return carry

        def pair_body(cp, carry):
            carry = half_chunk(2 * cp, 0, carry)
            carry = half_chunk(2 * cp + 1, 1, carry)
            return carry

        npairs = (nch + 1) // 2
        z16 = jnp.zeros((16,), jnp.float32)
        fin = lax.fori_loop(0, npairs, pair_body,
                            (jnp.float32(0.0), z16, z16, z16, z16, z16, z16,
                             z16, z16))
        ci = fin[0].astype(jnp.int32)
        msg_loc[ci, pl.ds(0, 16)] = fin[1]
        msg_loc[ci, pl.ds(16, 16)] = fin[2]
        msg_loc[ci, pl.ds(32, 16)] = fin[3]
        msg_loc[ci, pl.ds(48, 16)] = fin[4]
        msg_loc[ci, pl.ds(64, 16)] = fin[5]
        msg_loc[ci, pl.ds(80, 16)] = fin[6]
        msg_loc[ci, pl.ds(96, 16)] = fin[7]
        msg_loc[ci, pl.ds(112, 16)] = fin[8]
        wait_gather(0)
        pltpu.sync_copy(msg_loc, msg_hbm.at[pl.ds(node_base, _NLOC)])
        pltpu.sync_copy(ssum_loc, ssum_hbm.at[pl.ds(node_base, _NLOC)])

    return level_sc


@functools.lru_cache(maxsize=None)
def _level_sc_cached():
    return _make_level_sc()


_IC = 64


def _make_init_sc():
    mesh = plsc.VectorSubcoreMesh(core_axis_name="c", subcore_axis_name="s")

    @functools.partial(
        pl.kernel,
        out_type=[
            jax.ShapeDtypeStruct((_NP, _D), jnp.float32),
            jax.ShapeDtypeStruct((2, _NP), jnp.float32),
        ],
        mesh=mesh,
        compiler_params=pltpu.CompilerParams(needs_layout_passes=False),
        scratch_types=[
            pltpu.VMEM((_IC,), jnp.int32),
            pltpu.VMEM((16,), jnp.int32),
            pltpu.VMEM((_IC, _D), jnp.float32),
            pltpu.VMEM((_IC, _D), jnp.float32),
            pltpu.VMEM((128,), jnp.int32),
            pltpu.VMEM((128,), jnp.float32),
            pltpu.VMEM((16,), jnp.float32),
            pltpu.VMEM_SHARED((_NP,), jnp.float32),
            pltpu.SemaphoreType.DMA,
        ],
    )
    def init_sc(x0_hbm, x1_hbm, dep_hbm, temb_hbm, aemb_hbm, demb_hbm,
                srce_hbm, zn_hbm, on_hbm, hprev_hbm, lcnt_hbm,
                idx_c, idx_t, A, B, se_c, ones_v, ones_t, cnt_sh, sem):
        cid = lax.axis_index("c")
        sid = lax.axis_index("s")
        wid = cid * 16 + sid
        node_base = pl.multiple_of(wid * _NLOC, _NLOC)

        @pl.when(sid == 0)
        def _zero():
            pltpu.sync_copy(zn_hbm, cnt_sh)

        pltpu.sync_copy(on_hbm, ones_v)
        pltpu.sync_copy(on_hbm.at[pl.ds(0, 16)], ones_t)

        for c in range(_NLOC // _IC):
            nb = node_base + c * _IC
            sl = pl.ds(nb, _IC)
            pltpu.sync_copy(x0_hbm.at[sl], idx_c)
            pltpu.async_copy(temb_hbm.at[idx_c], A, sem).wait()
            pltpu.sync_copy(x1_hbm.at[sl], idx_c)
            pltpu.async_copy(aemb_hbm.at[idx_c], B, sem).wait()

            def add_row(r, carry):
                for j in range(_D // 16):
                    cs = pl.ds(j * 16, 16)
                    A[r, cs] = A[r, cs] + B[r, cs]
                return carry

            lax.fori_loop(0, _IC, add_row, 0)
            pltpu.sync_copy(dep_hbm.at[sl], idx_c)
            for j in range(_IC // 16):
                cs = pl.ds(j * 16, 16)
                idx_c[cs] = jnp.minimum(idx_c[cs], _MAXD)
            pltpu.async_copy(demb_hbm.at[idx_c], B, sem).wait()
            lax.fori_loop(0, _IC, add_row, 0)
            pltpu.sync_copy(A, hprev_hbm.at[sl])

        plsc.subcore_barrier()
        estart = wid * (_E // _NW)

        def cnt_body(ch, carry):
            base = pl.multiple_of(estart + ch * 128, 8)
            pltpu.sync_copy(srce_hbm.at[pl.ds(base, 128)], se_c)
            pltpu.sync_copy(ones_v, cnt_sh.at[se_c], add=True)
            return carry

        nfull = (_E // _NW) // 128
        lax.fori_loop(0, nfull, cnt_body, 0)
        tbase = pl.multiple_of(estart + nfull * 128, 8)
        pltpu.sync_copy(srce_hbm.at[pl.ds(tbase, 16)], idx_t)
        pltpu.sync_copy(ones_t, cnt_sh.at[idx_t], add=True)
        plsc.subcore_barrier()

        @pl.when(sid == 0)
        def _out():
            pltpu.sync_copy(cnt_sh, lcnt_hbm.at[cid])

    return init_sc


@functools.lru_cache(maxsize=None)
def _init_sc_cached():
    return _make_init_sc()


def kernel(x, node_depth, edge_index, edge_attr, edge_masks, batch,
           type_emb, attr_emb, depth_emb, edge_W, edge_b, attn_W, attn_b,
           Wih, Whh, bih, bhh):
    src, dst = edge_index[0], edge_index[1]

    order = jnp.argsort(dst)
    src_s = src[order].astype(jnp.int32)
    dst_s = dst[order].astype(jnp.int32)
    ea_s = edge_attr[order]
    mpow = (jnp.int32(1) << jnp.arange(8, dtype=jnp.int32))[:, None]
    mpack = jnp.sum(edge_masks.astype(jnp.int32) * mpow, axis=0,
                    dtype=jnp.int32)
    mp_s = mpack[order]
    row_ptr = jnp.searchsorted(dst_s, jnp.arange(_NP + 1, dtype=jnp.int32),
                               ).astype(jnp.int32)
    deg2d = (row_ptr[1:] - row_ptr[:-1]).reshape(_NP, 1)
    eb = row_ptr[jnp.arange(_NW + 1) * _NLOC]
    eb = jnp.pad(eb, (0, 15)).astype(jnp.int32)
    eb_levs = [eb.at[40].set(lev) for lev in range(_NLEVELS)]

    padE = _EP - _E
    src_p = jnp.pad(src_s, (0, padE))
    dst_p = jnp.pad(dst_s, (0, padE), constant_values=_NP - 1)
    mp_p = jnp.pad(mp_s, (0, padE))
    ea0 = jnp.pad(ea_s[:, 0], (0, padE)).reshape(_ER, 128)
    ea1 = jnp.pad(ea_s[:, 1], (0, padE)).reshape(_ER, 128)

    x0p = jnp.pad(x[:, 0], (0, _NP - _N)).astype(jnp.int32)
    x1p = jnp.pad(x[:, 1], (0, _NP - _N)).astype(jnp.int32)
    depp = jnp.pad(node_depth, (0, _NP - _N)).astype(jnp.int32)
    zn = jnp.zeros((_NP,), jnp.float32)
    on = jnp.ones((128,), jnp.float32)
    init_sc = _init_sc_cached()
    h_prev, lcnt = init_sc(x0p, x1p, depp, type_emb, attr_emb, depth_emb,
                           src_p, zn, on)
    lc0 = lcnt[0].reshape(_NP, 1)
    lc1 = lcnt[1].reshape(_NP, 1)

    w2s = attn_W[:, _D:, 0]
    ew2 = jnp.einsum('lcd,ld->lc', edge_W, w2s)
    g_all = _coef(ew2, ea0, ea1)
    g2 = g_all.reshape(2, _EP)
    z2 = jnp.zeros((_NLOC, _D), jnp.float32)
    z1 = jnp.zeros((_NLOC,), jnp.float32)
    level_sc = _level_sc_cached()

    batch2d = jnp.pad(batch, (0, _NP - _N),
                      constant_values=_NGRAPHS).astype(jnp.int32).reshape(_NP, 1)

    h_layers = []
    for l in range(_NLAYERS):
        w2 = attn_W[l, _D:, :]
        wih_t = Wih[l].T
        whh_t = Whh[l].T
        bih2 = bih[l].reshape(1, 3 * _D)
        bhh2 = bhh[l].reshape(1, 3 * _D)
        gi, h, q = _prep(h_prev, wih_t, bih2, bhh2, w2, deg2d)
        for lev in range(_NLEVELS):
            msg_raw, ssum = level_sc(q.reshape(_NP), h, src_p, dst_p,
                                     g2[l], mp_p, eb_levs[lev], z2, z1)
            h, q = _gru(gi, h, h_prev, msg_raw, ssum.reshape(_NP, 1),
                        whh_t, bhh2, w2)
        h_layers.append(h)
        h_prev = h

    H = jnp.concatenate(h_layers, axis=1)
    return _pool(H, batch2d, lc0, lc1)
